# Initial kernel scaffold; baseline (speedup 1.0000x reference)
import jax
import jax.numpy as jnp
from jax.experimental import pallas as pl


def kernel(inputs, A, mask_item, embedding, r_embed, trans_M, h_list, t_list, r_list, A_vals, a_0, a_1, a_2, a_3, w_3):
    raise NotImplementedError("write your pallas kernel here")


# speedup vs baseline: 3.8288x; 3.8288x over previous
import functools

import jax
import jax.numpy as jnp
from jax import lax
from jax.experimental import pallas as pl
from jax.experimental.pallas import tpu as pltpu
from jax.experimental.pallas import tpu_sc as plsc

N_NODE = 10000
N_REL = 8
E = 160000
DIM = 128
BS = 512
SQ = 20
ALPHA = 0.2

NC = 2
NS = 16
NW = NC * NS
L = 16

CH = 128
NCHUNK = E // CH
CPT = (NCHUNK + NW - 1) // NW
CPT1 = (NCHUNK + NS - 1) // NS
NKB = DIM // L

HALF = N_NODE // NC
ACC_ROWS = 5120
RCH = 40
NRCH = HALF // RCH
NZCH = ACC_ROWS // RCH
RCPT = NZCH // NS

_mesh = plsc.VectorSubcoreMesh(core_axis_name="c", subcore_axis_name="s")


def _worker_ids():
    c = lax.axis_index("c")
    s = lax.axis_index("s")
    return c, s, c * NS + s


def _lane_rotsum(x, lanes):
    for sh in (1, 2, 4, 8):
        idx = jnp.bitwise_and(lanes + sh, L - 1)
        x = x + x[idx]
    return x


def _lane_rotmax(x, lanes):
    for sh in (1, 2, 4, 8):
        idx = jnp.bitwise_and(lanes + sh, L - 1)
        x = jnp.maximum(x, x[idx])
    return x


def _zero_vmem(ref, nrows):
    def body(i, _):
        for k in range(ref.shape[1] // L):
            ref[i, pl.ds(k * L, L)] = jnp.zeros((L,), jnp.float32)
        return 0
    lax.fori_loop(0, nrows, body, 0)


@functools.partial(
    pl.kernel,
    out_type=jax.ShapeDtypeStruct((NC, HALF, DIM), jnp.float32),
    mesh=_mesh,
    scratch_types=[
        pltpu.VMEM((CH,), jnp.int32),
        pltpu.VMEM((CH,), jnp.int32),
        pltpu.VMEM((CH,), jnp.float32),
        pltpu.VMEM((CH, DIM), jnp.float32),
        pltpu.VMEM((RCH, DIM), jnp.float32),
        pltpu.VMEM_SHARED((ACC_ROWS, DIM), jnp.float32),
        pltpu.SemaphoreType.DMA,
    ],
)
def _spmm1(emb_hbm, hl_hbm, tl_hbm, av_hbm, out_hbm,
           tidx, hidx, vals, rows, zbuf, acc, sem):
    c, s, w = _worker_ids()
    _zero_vmem(zbuf, RCH)

    def zero(k, _):
        rc = s + NS * k
        pltpu.sync_copy(zbuf, acc.at[pl.ds(rc * RCH, RCH)])
        return 0
    lax.fori_loop(0, RCPT, zero, 0)
    plsc.subcore_barrier()
    lo = c * HALF
    dump = HALF + s

    def chunk(j, _):
        cid = s + NS * j

        @pl.when(cid < NCHUNK)
        def _():
            base = cid * CH
            pltpu.sync_copy(tl_hbm.at[pl.ds(base, CH)], tidx)
            pltpu.sync_copy(hl_hbm.at[pl.ds(base, CH)], hidx)
            pltpu.sync_copy(av_hbm.at[pl.ds(base, CH)], vals)
            for k in range(CH // L):
                hloc = hidx[pl.ds(k * L, L)] - lo
                ok = (hloc >= 0) & (hloc < HALF)
                hidx[pl.ds(k * L, L)] = jnp.where(ok, hloc, dump)
            pltpu.async_copy(emb_hbm.at[tidx], rows, sem).wait()

            def scale(g, _):
                v16 = vals[pl.ds(g * L, L)]
                for e16 in range(L):
                    v = v16[e16]
                    e = g * L + e16
                    for k in range(NKB):
                        rows[e, pl.ds(k * L, L)] = rows[e, pl.ds(k * L, L)] * v
                return 0
            lax.fori_loop(0, CH // L, scale, 0)
            pltpu.sync_copy(rows, acc.at[hidx], add=True)
        return 0
    lax.fori_loop(0, CPT1, chunk, 0)
    plsc.subcore_barrier()

    def wb(k, _):
        rc = s + NS * k

        @pl.when(rc < NRCH)
        def _():
            r0 = rc * RCH
            pltpu.sync_copy(acc.at[pl.ds(r0, RCH)], zbuf)
            pltpu.sync_copy(zbuf, out_hbm.at[c, pl.ds(r0, RCH)])
        return 0
    lax.fori_loop(0, RCPT, wb, 0)


def _relproj_body(ego_ref, w_ref, y_ref, t_ref):
    ego = ego_ref[...]
    for r in range(N_REL):
        yr = jnp.dot(ego, w_ref[r], preferred_element_type=jnp.float32)
        y_ref[r] = yr
        t_ref[r] = jnp.tanh(yr)


def _relproj(ego1, trans_M):
    M = 1000
    grid = (N_NODE // M,)
    return pl.pallas_call(
        _relproj_body,
        grid=grid,
        in_specs=[
            pl.BlockSpec((M, DIM), lambda i: (i, 0)),
            pl.BlockSpec((N_REL, DIM, DIM), lambda i: (0, 0, 0)),
        ],
        out_specs=[
            pl.BlockSpec((N_REL, M, DIM), lambda i: (0, i, 0)),
            pl.BlockSpec((N_REL, M, DIM), lambda i: (0, i, 0)),
        ],
        out_shape=[
            jax.ShapeDtypeStruct((N_REL, N_NODE, DIM), jnp.float32),
            jax.ShapeDtypeStruct((N_REL, N_NODE, DIM), jnp.float32),
        ],
    )(ego1, trans_M)


@functools.partial(
    pl.kernel,
    out_type=[
        jax.ShapeDtypeStruct((E,), jnp.float32),
        jax.ShapeDtypeStruct((NW, 8, L), jnp.float32),
    ],
    mesh=_mesh,
    scratch_types=[
        pltpu.VMEM((CH,), jnp.int32),
        pltpu.VMEM((CH,), jnp.int32),
        pltpu.VMEM((CH,), jnp.int32),
        pltpu.VMEM((CH,), jnp.int32),
        pltpu.VMEM((CH,), jnp.int32),
        pltpu.VMEM((CH, DIM), jnp.float32),
        pltpu.VMEM((CH, DIM), jnp.float32),
        pltpu.VMEM((L,), jnp.float32),
        pltpu.VMEM((CH,), jnp.float32),
        pltpu.VMEM((8, L), jnp.float32),
        pltpu.SemaphoreType.DMA,
        pltpu.SemaphoreType.DMA,
    ],
)
def _edgescore(y_hbm, t_hbm, hl_hbm, tl_hbm, rl_hbm, sv_hbm,
               v_hbm, mx_hbm,
               hidx, tidx, ridx, hgat, tgat, yrows, trows, svv, vbuf,
               mrow, sem1, sem2):
    c, s, w = _worker_ids()
    pltpu.sync_copy(sv_hbm, svv)
    lanes = jnp.arange(L, dtype=jnp.int32)
    mrow[0, pl.ds(0, L)] = jnp.full((L,), -3e38, jnp.float32)

    def chunk(j, _):
        cid = w + NW * j

        @pl.when(cid < NCHUNK)
        def do():
            base = cid * CH
            pltpu.sync_copy(hl_hbm.at[pl.ds(base, CH)], hidx)
            pltpu.sync_copy(tl_hbm.at[pl.ds(base, CH)], tidx)
            pltpu.sync_copy(rl_hbm.at[pl.ds(base, CH)], ridx)
            for k in range(CH // L):
                rr = ridx[pl.ds(k * L, L)] * N_NODE
                hgat[pl.ds(k * L, L)] = rr + hidx[pl.ds(k * L, L)]
                tgat[pl.ds(k * L, L)] = rr + tidx[pl.ds(k * L, L)]
            cp1 = pltpu.async_copy(y_hbm.at[hgat], yrows, sem1)
            cp2 = pltpu.async_copy(t_hbm.at[tgat], trows, sem2)
            cp1.wait()
            cp2.wait()

            def group(g, m):
                vvec = jnp.zeros((L,), jnp.float32)
                for e16 in range(L):
                    e = g * L + e16
                    acc = trows[e, pl.ds(0, L)] * yrows[e, pl.ds(0, L)]
                    for k in range(1, NKB):
                        acc = acc + trows[e, pl.ds(k * L, L)] * yrows[e, pl.ds(k * L, L)]
                    accs = _lane_rotsum(acc, lanes)
                    vvec = jnp.where(lanes == e16, accs, vvec)
                r16 = ridx[pl.ds(g * L, L)]
                svvec = svv[pl.ds(0, L)]
                sv = jnp.zeros((L,), jnp.float32)
                for r in range(N_REL):
                    sv = jnp.where(r16 == r, svvec[r], sv)
                vvec = vvec + sv
                vbuf[pl.ds(g * L, L)] = vvec
                return jnp.maximum(m, vvec)
            m = lax.fori_loop(0, CH // L, group, mrow[0, pl.ds(0, L)])
            mrow[0, pl.ds(0, L)] = m
            pltpu.sync_copy(vbuf, v_hbm.at[pl.ds(base, CH)])
        return 0
    lax.fori_loop(0, CPT, chunk, 0)
    m = mrow[0, pl.ds(0, L)]
    for i in range(8):
        mrow[i, pl.ds(0, L)] = m
    pltpu.sync_copy(mrow, mx_hbm.at[w])


def _expv_body(v_ref, mx_ref, ex_ref):
    m = jnp.max(mx_ref[...])
    ex_ref[...] = jnp.exp(v_ref[...] - m)


def _expv(v2d, mx):
    return pl.pallas_call(
        _expv_body,
        grid=(1,),
        in_specs=[
            pl.BlockSpec((E // DIM, DIM), lambda i: (0, 0)),
            pl.BlockSpec((NW * 8, L), lambda i: (0, 0)),
        ],
        out_specs=pl.BlockSpec((E // DIM, DIM), lambda i: (0, 0)),
        out_shape=jax.ShapeDtypeStruct((E // DIM, DIM), jnp.float32),
    )(v2d, mx)


@functools.partial(
    pl.kernel,
    out_type=[
        jax.ShapeDtypeStruct((NC, HALF, DIM), jnp.float32),
        jax.ShapeDtypeStruct((NC, HALF, DIM), jnp.float32),
    ],
    mesh=_mesh,
    scratch_types=[
        pltpu.VMEM((CH,), jnp.int32),
        pltpu.VMEM((CH,), jnp.int32),
        pltpu.VMEM((CH,), jnp.float32),
        pltpu.VMEM((CH, DIM), jnp.float32),
        pltpu.VMEM((CH, DIM), jnp.float32),
        pltpu.VMEM((RCH, DIM), jnp.float32),
        pltpu.VMEM((RCH, DIM), jnp.float32),
        pltpu.VMEM_SHARED((ACC_ROWS, DIM), jnp.float32),
        pltpu.VMEM_SHARED((ACC_ROWS, DIM), jnp.float32),
        pltpu.SemaphoreType.DMA,
    ],
)
def _hop2(ego1_hbm, hl_hbm, tl_hbm, ex_hbm,
          e2_hbm, den_hbm,
          hidx, tidx, vbuf, rows, denrows, zbuf, zbufd, acc, dacc, sem):
    c, s, w = _worker_ids()
    _zero_vmem(zbuf, RCH)
    _zero_vmem(zbufd, RCH)
    _zero_vmem(denrows, CH)

    def zero(k, _):
        rc = s + NS * k
        pltpu.sync_copy(zbuf, acc.at[pl.ds(rc * RCH, RCH)])
        pltpu.sync_copy(zbufd, dacc.at[pl.ds(rc * RCH, RCH)])
        return 0
    lax.fori_loop(0, RCPT, zero, 0)
    plsc.subcore_barrier()
    lo = c * HALF
    dump = HALF + s

    def chunk(j, _):
        cid = s + NS * j

        @pl.when(cid < NCHUNK)
        def _():
            base = cid * CH
            pltpu.sync_copy(ex_hbm.at[pl.ds(base, CH)], vbuf)
            pltpu.sync_copy(hl_hbm.at[pl.ds(base, CH)], hidx)
            pltpu.sync_copy(tl_hbm.at[pl.ds(base, CH)], tidx)
            for k in range(CH // L):
                hloc = hidx[pl.ds(k * L, L)] - lo
                ok = (hloc >= 0) & (hloc < HALF)
                hidx[pl.ds(k * L, L)] = jnp.where(ok, hloc, dump)
            pltpu.async_copy(ego1_hbm.at[tidx], rows, sem).wait()

            def scale(g, _):
                x16 = vbuf[pl.ds(g * L, L)]
                for e16 in range(L):
                    x = x16[e16]
                    e = g * L + e16
                    xv = jnp.full((L,), x, jnp.float32)
                    denrows[e, pl.ds(0, L)] = xv
                    for k in range(NKB):
                        rows[e, pl.ds(k * L, L)] = rows[e, pl.ds(k * L, L)] * xv
                return 0
            lax.fori_loop(0, CH // L, scale, 0)
            pltpu.sync_copy(rows, acc.at[hidx], add=True)
            pltpu.sync_copy(denrows, dacc.at[hidx], add=True)
        return 0
    lax.fori_loop(0, CPT1, chunk, 0)
    plsc.subcore_barrier()

    def wb(k, _):
        rc = s + NS * k

        @pl.when(rc < NRCH)
        def _():
            r0 = rc * RCH
            pltpu.sync_copy(acc.at[pl.ds(r0, RCH)], zbuf)
            pltpu.sync_copy(zbuf, e2_hbm.at[c, pl.ds(r0, RCH)])
            pltpu.sync_copy(dacc.at[pl.ds(r0, RCH)], zbufd)
            pltpu.sync_copy(zbufd, den_hbm.at[c, pl.ds(r0, RCH)])
        return 0
    lax.fori_loop(0, RCPT, wb, 0)


def _normkg_body(ego_ref, e2_ref, den_ref, kg_ref):
    d = den_ref[...][:, 0:1]
    d = jnp.where(d == 0.0, 1.0, d)
    kg_ref[...] = (ego_ref[...] + e2_ref[...] / d) * 0.5


def _normkg(ego1, e2, den):
    M = 1000
    return pl.pallas_call(
        _normkg_body,
        grid=(N_NODE // M,),
        in_specs=[
            pl.BlockSpec((M, DIM), lambda i: (i, 0)),
            pl.BlockSpec((M, DIM), lambda i: (i, 0)),
            pl.BlockSpec((M, DIM), lambda i: (i, 0)),
        ],
        out_specs=pl.BlockSpec((M, DIM), lambda i: (i, 0)),
        out_shape=jax.ShapeDtypeStruct((N_NODE, DIM), jnp.float32),
    )(ego1, e2, den)


NIDX = BS * SQ
IPT = NIDX // NW
ICH = 64


@functools.partial(
    pl.kernel,
    out_type=[
        jax.ShapeDtypeStruct((NIDX, DIM), jnp.float32),
        jax.ShapeDtypeStruct((NIDX, DIM), jnp.float32),
    ],
    mesh=_mesh,
    scratch_types=[
        pltpu.VMEM((ICH,), jnp.int32),
        pltpu.VMEM((ICH, DIM), jnp.float32),
        pltpu.VMEM((ICH, DIM), jnp.float32),
        pltpu.SemaphoreType.DMA,
        pltpu.SemaphoreType.DMA,
    ],
)
def _assemble(emb_hbm, kg_hbm,
              idx_hbm, hout_hbm, kgout_hbm,
              idxv, erows, grows, sem1, sem2):
    c, s, w = _worker_ids()

    def chunk(i, _):
        base = w * IPT + i * ICH
        pltpu.sync_copy(idx_hbm.at[pl.ds(base, ICH)], idxv)
        cp1 = pltpu.async_copy(emb_hbm.at[idxv], erows, sem1)
        cp2 = pltpu.async_copy(kg_hbm.at[idxv], grows, sem2)
        cp1.wait()
        cp2.wait()
        pltpu.sync_copy(erows, hout_hbm.at[pl.ds(base, ICH)])
        pltpu.sync_copy(grows, kgout_hbm.at[pl.ds(base, ICH)])
        return 0
    lax.fori_loop(0, IPT // ICH, chunk, 0)


BBLK = 8


def _dense_body(h_ref, kg_ref, a_ref, av_ref, w3_ref, out_ref):
    h = h_ref[...]
    kg = kg_ref[...]
    Ab = a_ref[...]
    av = av_ref[...]
    w3 = w3_ref[...]

    P = h[:, :, None, :] * h[:, None, :, :]
    P = jnp.where(P > 0, P, ALPHA * P)
    e4 = jnp.dot(P.reshape(BBLK * SQ * SQ, DIM), av,
                 preferred_element_type=jnp.float32)
    e4 = e4.reshape(BBLK, SQ, SQ, 4)

    big = jnp.float32(-9e15)
    al = jnp.full((BBLK, SQ, SQ), big, jnp.float32)
    for k in range(4):
        al = jnp.where(Ab == (k + 1), e4[..., k], al)
    al = al - jnp.max(al, axis=-1, keepdims=True)
    al = jnp.exp(al)
    al = al / jnp.sum(al, axis=-1, keepdims=True)

    hid = []
    for b in range(BBLK):
        hid.append(jnp.dot(al[b], h[b], preferred_element_type=jnp.float32))
    hidden = jnp.stack(hid)

    out = jnp.dot(hidden.reshape(BBLK * SQ, DIM), w3[:DIM],
                  preferred_element_type=jnp.float32)
    out = out + jnp.dot(kg.reshape(BBLK * SQ, DIM), w3[DIM:],
                        preferred_element_type=jnp.float32)
    out_ref[...] = jnp.tanh(out).reshape(BBLK, SQ, DIM)


def _dense(hrows, kgrows, A, av, w3):
    grid = (BS // BBLK,)
    return pl.pallas_call(
        _dense_body,
        grid=grid,
        in_specs=[
            pl.BlockSpec((BBLK, SQ, DIM), lambda i: (i, 0, 0)),
            pl.BlockSpec((BBLK, SQ, DIM), lambda i: (i, 0, 0)),
            pl.BlockSpec((BBLK, SQ, SQ), lambda i: (i, 0, 0)),
            pl.BlockSpec((DIM, 4), lambda i: (0, 0)),
            pl.BlockSpec((2 * DIM, DIM), lambda i: (0, 0)),
        ],
        out_specs=pl.BlockSpec((BBLK, SQ, DIM), lambda i: (i, 0, 0)),
        out_shape=jax.ShapeDtypeStruct((BS, SQ, DIM), jnp.float32),
    )(hrows, kgrows, A, av, w3)


def _xla_rest(inputs, A, emb, r_embed, trans_M, hl, tl, rl, A_vals,
              a_0, a_1, a_2, a_3, w_3, ego1, v=None, mx=None, kg=None):
    if v is None:
        Y = jnp.einsum('nd,rde->rne', ego1, trans_M)
        T = jnp.tanh(Y)
        sv = jnp.sum(r_embed, axis=1)
        yf = Y.reshape(N_REL * N_NODE, DIM)
        tf = T.reshape(N_REL * N_NODE, DIM)
        v = jnp.sum(tf[rl * N_NODE + tl] * yf[rl * N_NODE + hl], axis=1) + sv[rl]
    if kg is None:
        gm = jnp.max(v) if mx is None else jnp.max(mx)
        ex = jnp.exp(v - gm)
        den = jax.ops.segment_sum(ex, hl, num_segments=N_NODE)
        e2num = jax.ops.segment_sum(ex[:, None] * ego1[tl], hl, num_segments=N_NODE)
        dv = jnp.where(den == 0, 1., den)
        kg = (ego1 + e2num / dv[:, None]) * 0.5
    idx = inputs.astype(jnp.int32).reshape(-1)
    h = emb[idx].reshape(BS, SQ, DIM)
    kgr = kg[idx].reshape(BS, SQ, DIM)
    ai = h[:, :, None, :] * h[:, None, :, :]
    ai = jnp.where(ai > 0, ai, ALPHA * ai)
    av4 = jnp.concatenate([a_0, a_1, a_2, a_3], axis=1)
    e4 = (ai.reshape(-1, DIM) @ av4).reshape(BS, SQ, SQ, 4)
    al = jnp.full((BS, SQ, SQ), -9e15)
    for k in range(4):
        al = jnp.where(A == k + 1, e4[..., k], al)
    al = jax.nn.softmax(al, axis=-1)
    hid = jnp.einsum('bij,bjd->bid', al, h)
    return jnp.tanh(jnp.concatenate([hid, kgr], axis=-1) @ w_3)


def kernel(inputs, A, mask_item, embedding, r_embed, trans_M, h_list,
           t_list, r_list, A_vals, a_0, a_1, a_2, a_3, w_3):
    hl = h_list.astype(jnp.int32)
    tl = t_list.astype(jnp.int32)
    rl = r_list.astype(jnp.int32)
    emb = embedding.astype(jnp.float32)

    ego1 = _spmm1(emb, hl, tl, A_vals).reshape(N_NODE, DIM)
    Y, T = _relproj(ego1, trans_M)
    yflat = Y.reshape(N_REL * N_NODE, DIM)
    tflat = T.reshape(N_REL * N_NODE, DIM)
    svec = jnp.zeros((L,), jnp.float32).at[:N_REL].set(jnp.sum(r_embed, axis=1))
    v, mx = _edgescore(yflat, tflat, hl, tl, rl, svec)
    ex = _expv(v.reshape(E // DIM, DIM), mx.reshape(NW * 8, L)).reshape(E)
    e2, den = _hop2(ego1, hl, tl, ex)
    kg = _normkg(ego1, e2.reshape(N_NODE, DIM), den.reshape(N_NODE, DIM))
    idx = inputs.astype(jnp.int32).reshape(-1)
    hrows, kgrows = _assemble(emb, kg, idx)
    av = jnp.concatenate([a_0, a_1, a_2, a_3], axis=1)
    return _dense(hrows.reshape(BS, SQ, DIM), kgrows.reshape(BS, SQ, DIM),
                  A.astype(jnp.int32), av, w_3)
    Y, T = _relproj(ego1, trans_M)
    yflat = Y.reshape(N_REL * N_NODE, DIM)
    tflat = T.reshape(N_REL * N_NODE, DIM)

    svec = jnp.zeros((L,), jnp.float32).at[:N_REL].set(jnp.sum(r_embed, axis=1))
    v, mx = _edgescore(yflat, tflat, hl, tl, rl, svec)
    e2, den = _hop2(ego1, hl, tl, v, mx.reshape(NW * 8, L))

    kg = _normkg(ego1, e2.reshape(N_NODE, DIM), den.reshape(N_NODE, DIM))
    idx = inputs.astype(jnp.int32).reshape(-1)
    hrows, kgrows = _assemble(emb, kg, idx)

    av = jnp.concatenate([a_0, a_1, a_2, a_3], axis=1)
    out = _dense(hrows.reshape(BS, SQ, DIM), kgrows.reshape(BS, SQ, DIM),
                 A.astype(jnp.int32), av, w_3)
    return out


# speedup vs baseline: 3.8379x; 1.0024x over previous
import functools

import jax
import jax.numpy as jnp
from jax import lax
from jax.experimental import pallas as pl
from jax.experimental.pallas import tpu as pltpu
from jax.experimental.pallas import tpu_sc as plsc

N_NODE = 10000
N_REL = 8
E = 160000
DIM = 128
BS = 512
SQ = 20
ALPHA = 0.2

NC = 2
NS = 16
NW = NC * NS
L = 16

CH = 128
NCHUNK = E // CH
CPT = (NCHUNK + NW - 1) // NW
CPT1 = (NCHUNK + NS - 1) // NS
NKB = DIM // L

HALF = N_NODE // NC
ACC_ROWS = 5120
RCH = 40
NRCH = HALF // RCH
NZCH = ACC_ROWS // RCH
RCPT = NZCH // NS

_mesh = plsc.VectorSubcoreMesh(core_axis_name="c", subcore_axis_name="s")


def _worker_ids():
    c = lax.axis_index("c")
    s = lax.axis_index("s")
    return c, s, c * NS + s


def _lane_rotsum(x, lanes):
    for sh in (1, 2, 4, 8):
        idx = jnp.bitwise_and(lanes + sh, L - 1)
        x = x + x[idx]
    return x


def _lane_rotmax(x, lanes):
    for sh in (1, 2, 4, 8):
        idx = jnp.bitwise_and(lanes + sh, L - 1)
        x = jnp.maximum(x, x[idx])
    return x


def _zero_vmem(ref, nrows):
    def body(i, _):
        for k in range(ref.shape[1] // L):
            ref[i, pl.ds(k * L, L)] = jnp.zeros((L,), jnp.float32)
        return 0
    lax.fori_loop(0, nrows, body, 0)


@functools.partial(
    pl.kernel,
    out_type=jax.ShapeDtypeStruct((NC, HALF, DIM), jnp.float32),
    mesh=_mesh,
    scratch_types=[
        pltpu.VMEM((CH,), jnp.int32),
        pltpu.VMEM((CH,), jnp.int32),
        pltpu.VMEM((CH,), jnp.float32),
        pltpu.VMEM((CH, DIM), jnp.float32),
        pltpu.VMEM((RCH, DIM), jnp.float32),
        pltpu.VMEM_SHARED((ACC_ROWS, DIM), jnp.float32),
        pltpu.SemaphoreType.DMA,
    ],
)
def _spmm1(emb_hbm, hl_hbm, tl_hbm, av_hbm, out_hbm,
           tidx, hidx, vals, rows, zbuf, acc, sem):
    c, s, w = _worker_ids()
    _zero_vmem(zbuf, RCH)

    def zero(k, _):
        rc = s + NS * k
        pltpu.sync_copy(zbuf, acc.at[pl.ds(rc * RCH, RCH)])
        return 0
    lax.fori_loop(0, RCPT, zero, 0)
    plsc.subcore_barrier()
    lo = c * HALF
    dump = HALF + s

    def chunk(j, _):
        cid = s + NS * j

        @pl.when(cid < NCHUNK)
        def _():
            base = cid * CH
            pltpu.sync_copy(tl_hbm.at[pl.ds(base, CH)], tidx)
            pltpu.sync_copy(hl_hbm.at[pl.ds(base, CH)], hidx)
            pltpu.sync_copy(av_hbm.at[pl.ds(base, CH)], vals)
            for k in range(CH // L):
                hloc = hidx[pl.ds(k * L, L)] - lo
                ok = (hloc >= 0) & (hloc < HALF)
                hidx[pl.ds(k * L, L)] = jnp.where(ok, hloc, dump)
            pltpu.async_copy(emb_hbm.at[tidx], rows, sem).wait()

            @plsc.parallel_loop(0, CH // L, unroll=2)
            def scale(g):
                v16 = vals[pl.ds(g * L, L)]
                for e16 in range(L):
                    v = v16[e16]
                    e = g * L + e16
                    for k in range(NKB):
                        rows[e, pl.ds(k * L, L)] = rows[e, pl.ds(k * L, L)] * v
            pltpu.sync_copy(rows, acc.at[hidx], add=True)
        return 0
    lax.fori_loop(0, CPT1, chunk, 0)
    plsc.subcore_barrier()

    def wb(k, _):
        rc = s + NS * k

        @pl.when(rc < NRCH)
        def _():
            r0 = rc * RCH
            pltpu.sync_copy(acc.at[pl.ds(r0, RCH)], zbuf)
            pltpu.sync_copy(zbuf, out_hbm.at[c, pl.ds(r0, RCH)])
        return 0
    lax.fori_loop(0, RCPT, wb, 0)


def _relproj_body(ego_ref, w_ref, y_ref, t_ref):
    ego = ego_ref[...]
    for r in range(N_REL):
        yr = jnp.dot(ego, w_ref[r], preferred_element_type=jnp.float32)
        y_ref[r] = yr
        t_ref[r] = jnp.tanh(yr)


def _relproj(ego1, trans_M):
    M = 1000
    grid = (N_NODE // M,)
    return pl.pallas_call(
        _relproj_body,
        grid=grid,
        in_specs=[
            pl.BlockSpec((M, DIM), lambda i: (i, 0)),
            pl.BlockSpec((N_REL, DIM, DIM), lambda i: (0, 0, 0)),
        ],
        out_specs=[
            pl.BlockSpec((N_REL, M, DIM), lambda i: (0, i, 0)),
            pl.BlockSpec((N_REL, M, DIM), lambda i: (0, i, 0)),
        ],
        out_shape=[
            jax.ShapeDtypeStruct((N_REL, N_NODE, DIM), jnp.float32),
            jax.ShapeDtypeStruct((N_REL, N_NODE, DIM), jnp.float32),
        ],
    )(ego1, trans_M)


@functools.partial(
    pl.kernel,
    out_type=[
        jax.ShapeDtypeStruct((E,), jnp.float32),
        jax.ShapeDtypeStruct((NW, 8, L), jnp.float32),
    ],
    mesh=_mesh,
    scratch_types=[
        pltpu.VMEM((CH,), jnp.int32),
        pltpu.VMEM((CH,), jnp.int32),
        pltpu.VMEM((CH,), jnp.int32),
        pltpu.VMEM((CH,), jnp.int32),
        pltpu.VMEM((CH,), jnp.int32),
        pltpu.VMEM((CH, DIM), jnp.float32),
        pltpu.VMEM((CH, DIM), jnp.float32),
        pltpu.VMEM((L,), jnp.float32),
        pltpu.VMEM((CH,), jnp.float32),
        pltpu.VMEM((8, L), jnp.float32),
        pltpu.SemaphoreType.DMA,
        pltpu.SemaphoreType.DMA,
    ],
)
def _edgescore(y_hbm, t_hbm, hl_hbm, tl_hbm, rl_hbm, sv_hbm,
               v_hbm, mx_hbm,
               hidx, tidx, ridx, hgat, tgat, yrows, trows, svv, vbuf,
               mrow, sem1, sem2):
    c, s, w = _worker_ids()
    pltpu.sync_copy(sv_hbm, svv)
    lanes = jnp.arange(L, dtype=jnp.int32)
    mrow[0, pl.ds(0, L)] = jnp.full((L,), -3e38, jnp.float32)

    def chunk(j, _):
        cid = w + NW * j

        @pl.when(cid < NCHUNK)
        def do():
            base = cid * CH
            pltpu.sync_copy(hl_hbm.at[pl.ds(base, CH)], hidx)
            pltpu.sync_copy(tl_hbm.at[pl.ds(base, CH)], tidx)
            pltpu.sync_copy(rl_hbm.at[pl.ds(base, CH)], ridx)
            for k in range(CH // L):
                rr = ridx[pl.ds(k * L, L)] * N_NODE
                hgat[pl.ds(k * L, L)] = rr + hidx[pl.ds(k * L, L)]
                tgat[pl.ds(k * L, L)] = rr + tidx[pl.ds(k * L, L)]
            cp1 = pltpu.async_copy(y_hbm.at[hgat], yrows, sem1)
            cp2 = pltpu.async_copy(t_hbm.at[tgat], trows, sem2)
            cp1.wait()
            cp2.wait()

            def group(g, m):
                vvec = jnp.zeros((L,), jnp.float32)
                for e16 in range(L):
                    e = g * L + e16
                    acc = trows[e, pl.ds(0, L)] * yrows[e, pl.ds(0, L)]
                    for k in range(1, NKB):
                        acc = acc + trows[e, pl.ds(k * L, L)] * yrows[e, pl.ds(k * L, L)]
                    accs = _lane_rotsum(acc, lanes)
                    vvec = jnp.where(lanes == e16, accs, vvec)
                r16 = ridx[pl.ds(g * L, L)]
                svvec = svv[pl.ds(0, L)]
                sv = jnp.zeros((L,), jnp.float32)
                for r in range(N_REL):
                    sv = jnp.where(r16 == r, svvec[r], sv)
                vvec = vvec + sv
                vbuf[pl.ds(g * L, L)] = vvec
                return jnp.maximum(m, vvec)
            m0 = mrow[0, pl.ds(0, L)]
            m = plsc.parallel_loop(0, CH // L, unroll=2, carry=m0)(group)
            mrow[0, pl.ds(0, L)] = m
            pltpu.sync_copy(vbuf, v_hbm.at[pl.ds(base, CH)])
        return 0
    lax.fori_loop(0, CPT, chunk, 0)
    m = mrow[0, pl.ds(0, L)]
    for i in range(8):
        mrow[i, pl.ds(0, L)] = m
    pltpu.sync_copy(mrow, mx_hbm.at[w])


def _expv_body(v_ref, mx_ref, ex_ref):
    m = jnp.max(mx_ref[...])
    ex_ref[...] = jnp.exp(v_ref[...] - m)


def _expv(v2d, mx):
    return pl.pallas_call(
        _expv_body,
        grid=(1,),
        in_specs=[
            pl.BlockSpec((E // DIM, DIM), lambda i: (0, 0)),
            pl.BlockSpec((NW * 8, L), lambda i: (0, 0)),
        ],
        out_specs=pl.BlockSpec((E // DIM, DIM), lambda i: (0, 0)),
        out_shape=jax.ShapeDtypeStruct((E // DIM, DIM), jnp.float32),
    )(v2d, mx)


@functools.partial(
    pl.kernel,
    out_type=[
        jax.ShapeDtypeStruct((NC, HALF, DIM), jnp.float32),
        jax.ShapeDtypeStruct((NC, HALF, DIM), jnp.float32),
    ],
    mesh=_mesh,
    scratch_types=[
        pltpu.VMEM((CH,), jnp.int32),
        pltpu.VMEM((CH,), jnp.int32),
        pltpu.VMEM((CH,), jnp.float32),
        pltpu.VMEM((CH, DIM), jnp.float32),
        pltpu.VMEM((CH, DIM), jnp.float32),
        pltpu.VMEM((RCH, DIM), jnp.float32),
        pltpu.VMEM((RCH, DIM), jnp.float32),
        pltpu.VMEM_SHARED((ACC_ROWS, DIM), jnp.float32),
        pltpu.VMEM_SHARED((ACC_ROWS, DIM), jnp.float32),
        pltpu.SemaphoreType.DMA,
    ],
)
def _hop2(ego1_hbm, hl_hbm, tl_hbm, ex_hbm,
          e2_hbm, den_hbm,
          hidx, tidx, vbuf, rows, denrows, zbuf, zbufd, acc, dacc, sem):
    c, s, w = _worker_ids()
    _zero_vmem(zbuf, RCH)
    _zero_vmem(zbufd, RCH)
    _zero_vmem(denrows, CH)

    def zero(k, _):
        rc = s + NS * k
        pltpu.sync_copy(zbuf, acc.at[pl.ds(rc * RCH, RCH)])
        pltpu.sync_copy(zbufd, dacc.at[pl.ds(rc * RCH, RCH)])
        return 0
    lax.fori_loop(0, RCPT, zero, 0)
    plsc.subcore_barrier()
    lo = c * HALF
    dump = HALF + s

    def chunk(j, _):
        cid = s + NS * j

        @pl.when(cid < NCHUNK)
        def _():
            base = cid * CH
            pltpu.sync_copy(ex_hbm.at[pl.ds(base, CH)], vbuf)
            pltpu.sync_copy(hl_hbm.at[pl.ds(base, CH)], hidx)
            pltpu.sync_copy(tl_hbm.at[pl.ds(base, CH)], tidx)
            for k in range(CH // L):
                hloc = hidx[pl.ds(k * L, L)] - lo
                ok = (hloc >= 0) & (hloc < HALF)
                hidx[pl.ds(k * L, L)] = jnp.where(ok, hloc, dump)
            pltpu.async_copy(ego1_hbm.at[tidx], rows, sem).wait()

            @plsc.parallel_loop(0, CH // L, unroll=2)
            def scale(g):
                x16 = vbuf[pl.ds(g * L, L)]
                for e16 in range(L):
                    x = x16[e16]
                    e = g * L + e16
                    xv = jnp.full((L,), x, jnp.float32)
                    denrows[e, pl.ds(0, L)] = xv
                    for k in range(NKB):
                        rows[e, pl.ds(k * L, L)] = rows[e, pl.ds(k * L, L)] * xv
            pltpu.sync_copy(rows, acc.at[hidx], add=True)
            pltpu.sync_copy(denrows, dacc.at[hidx], add=True)
        return 0
    lax.fori_loop(0, CPT1, chunk, 0)
    plsc.subcore_barrier()

    def wb(k, _):
        rc = s + NS * k

        @pl.when(rc < NRCH)
        def _():
            r0 = rc * RCH
            pltpu.sync_copy(acc.at[pl.ds(r0, RCH)], zbuf)
            pltpu.sync_copy(zbuf, e2_hbm.at[c, pl.ds(r0, RCH)])
            pltpu.sync_copy(dacc.at[pl.ds(r0, RCH)], zbufd)
            pltpu.sync_copy(zbufd, den_hbm.at[c, pl.ds(r0, RCH)])
        return 0
    lax.fori_loop(0, RCPT, wb, 0)


def _normkg_body(ego_ref, e2_ref, den_ref, kg_ref):
    d = den_ref[...][:, 0:1]
    d = jnp.where(d == 0.0, 1.0, d)
    kg_ref[...] = (ego_ref[...] + e2_ref[...] / d) * 0.5


def _normkg(ego1, e2, den):
    M = 1000
    return pl.pallas_call(
        _normkg_body,
        grid=(N_NODE // M,),
        in_specs=[
            pl.BlockSpec((M, DIM), lambda i: (i, 0)),
            pl.BlockSpec((M, DIM), lambda i: (i, 0)),
            pl.BlockSpec((M, DIM), lambda i: (i, 0)),
        ],
        out_specs=pl.BlockSpec((M, DIM), lambda i: (i, 0)),
        out_shape=jax.ShapeDtypeStruct((N_NODE, DIM), jnp.float32),
    )(ego1, e2, den)


NIDX = BS * SQ
IPT = NIDX // NW
ICH = 64


@functools.partial(
    pl.kernel,
    out_type=[
        jax.ShapeDtypeStruct((NIDX, DIM), jnp.float32),
        jax.ShapeDtypeStruct((NIDX, DIM), jnp.float32),
    ],
    mesh=_mesh,
    scratch_types=[
        pltpu.VMEM((ICH,), jnp.int32),
        pltpu.VMEM((ICH, DIM), jnp.float32),
        pltpu.VMEM((ICH, DIM), jnp.float32),
        pltpu.SemaphoreType.DMA,
        pltpu.SemaphoreType.DMA,
    ],
)
def _assemble(emb_hbm, kg_hbm,
              idx_hbm, hout_hbm, kgout_hbm,
              idxv, erows, grows, sem1, sem2):
    c, s, w = _worker_ids()

    def chunk(i, _):
        base = w * IPT + i * ICH
        pltpu.sync_copy(idx_hbm.at[pl.ds(base, ICH)], idxv)
        cp1 = pltpu.async_copy(emb_hbm.at[idxv], erows, sem1)
        cp2 = pltpu.async_copy(kg_hbm.at[idxv], grows, sem2)
        cp1.wait()
        cp2.wait()
        pltpu.sync_copy(erows, hout_hbm.at[pl.ds(base, ICH)])
        pltpu.sync_copy(grows, kgout_hbm.at[pl.ds(base, ICH)])
        return 0
    lax.fori_loop(0, IPT // ICH, chunk, 0)


BBLK = 8


def _dense_body(h_ref, kg_ref, a_ref, av_ref, w3_ref, out_ref):
    h = h_ref[...]
    kg = kg_ref[...]
    Ab = a_ref[...]
    av = av_ref[...]
    w3 = w3_ref[...]

    P = h[:, :, None, :] * h[:, None, :, :]
    P = jnp.where(P > 0, P, ALPHA * P)
    e4 = jnp.dot(P.reshape(BBLK * SQ * SQ, DIM), av,
                 preferred_element_type=jnp.float32)
    e4 = e4.reshape(BBLK, SQ, SQ, 4)

    big = jnp.float32(-9e15)
    al = jnp.full((BBLK, SQ, SQ), big, jnp.float32)
    for k in range(4):
        al = jnp.where(Ab == (k + 1), e4[..., k], al)
    al = al - jnp.max(al, axis=-1, keepdims=True)
    al = jnp.exp(al)
    al = al / jnp.sum(al, axis=-1, keepdims=True)

    hid = []
    for b in range(BBLK):
        hid.append(jnp.dot(al[b], h[b], preferred_element_type=jnp.float32))
    hidden = jnp.stack(hid)

    out = jnp.dot(hidden.reshape(BBLK * SQ, DIM), w3[:DIM],
                  preferred_element_type=jnp.float32)
    out = out + jnp.dot(kg.reshape(BBLK * SQ, DIM), w3[DIM:],
                        preferred_element_type=jnp.float32)
    out_ref[...] = jnp.tanh(out).reshape(BBLK, SQ, DIM)


def _dense(hrows, kgrows, A, av, w3):
    grid = (BS // BBLK,)
    return pl.pallas_call(
        _dense_body,
        grid=grid,
        in_specs=[
            pl.BlockSpec((BBLK, SQ, DIM), lambda i: (i, 0, 0)),
            pl.BlockSpec((BBLK, SQ, DIM), lambda i: (i, 0, 0)),
            pl.BlockSpec((BBLK, SQ, SQ), lambda i: (i, 0, 0)),
            pl.BlockSpec((DIM, 4), lambda i: (0, 0)),
            pl.BlockSpec((2 * DIM, DIM), lambda i: (0, 0)),
        ],
        out_specs=pl.BlockSpec((BBLK, SQ, DIM), lambda i: (i, 0, 0)),
        out_shape=jax.ShapeDtypeStruct((BS, SQ, DIM), jnp.float32),
    )(hrows, kgrows, A, av, w3)


def _xla_rest(inputs, A, emb, r_embed, trans_M, hl, tl, rl, A_vals,
              a_0, a_1, a_2, a_3, w_3, ego1, v=None, mx=None, kg=None):
    if v is None:
        Y = jnp.einsum('nd,rde->rne', ego1, trans_M)
        T = jnp.tanh(Y)
        sv = jnp.sum(r_embed, axis=1)
        yf = Y.reshape(N_REL * N_NODE, DIM)
        tf = T.reshape(N_REL * N_NODE, DIM)
        v = jnp.sum(tf[rl * N_NODE + tl] * yf[rl * N_NODE + hl], axis=1) + sv[rl]
    if kg is None:
        gm = jnp.max(v) if mx is None else jnp.max(mx)
        ex = jnp.exp(v - gm)
        den = jax.ops.segment_sum(ex, hl, num_segments=N_NODE)
        e2num = jax.ops.segment_sum(ex[:, None] * ego1[tl], hl, num_segments=N_NODE)
        dv = jnp.where(den == 0, 1., den)
        kg = (ego1 + e2num / dv[:, None]) * 0.5
    idx = inputs.astype(jnp.int32).reshape(-1)
    h = emb[idx].reshape(BS, SQ, DIM)
    kgr = kg[idx].reshape(BS, SQ, DIM)
    ai = h[:, :, None, :] * h[:, None, :, :]
    ai = jnp.where(ai > 0, ai, ALPHA * ai)
    av4 = jnp.concatenate([a_0, a_1, a_2, a_3], axis=1)
    e4 = (ai.reshape(-1, DIM) @ av4).reshape(BS, SQ, SQ, 4)
    al = jnp.full((BS, SQ, SQ), -9e15)
    for k in range(4):
        al = jnp.where(A == k + 1, e4[..., k], al)
    al = jax.nn.softmax(al, axis=-1)
    hid = jnp.einsum('bij,bjd->bid', al, h)
    return jnp.tanh(jnp.concatenate([hid, kgr], axis=-1) @ w_3)


def kernel(inputs, A, mask_item, embedding, r_embed, trans_M, h_list,
           t_list, r_list, A_vals, a_0, a_1, a_2, a_3, w_3):
    hl = h_list.astype(jnp.int32)
    tl = t_list.astype(jnp.int32)
    rl = r_list.astype(jnp.int32)
    emb = embedding.astype(jnp.float32)

    ego1 = _spmm1(emb, hl, tl, A_vals).reshape(N_NODE, DIM)
    Y, T = _relproj(ego1, trans_M)
    yflat = Y.reshape(N_REL * N_NODE, DIM)
    tflat = T.reshape(N_REL * N_NODE, DIM)
    svec = jnp.zeros((L,), jnp.float32).at[:N_REL].set(jnp.sum(r_embed, axis=1))
    v, mx = _edgescore(yflat, tflat, hl, tl, rl, svec)
    ex = _expv(v.reshape(E // DIM, DIM), mx.reshape(NW * 8, L)).reshape(E)
    e2, den = _hop2(ego1, hl, tl, ex)
    kg = _normkg(ego1, e2.reshape(N_NODE, DIM), den.reshape(N_NODE, DIM))
    idx = inputs.astype(jnp.int32).reshape(-1)
    hrows, kgrows = _assemble(emb, kg, idx)
    av = jnp.concatenate([a_0, a_1, a_2, a_3], axis=1)
    return _dense(hrows.reshape(BS, SQ, DIM), kgrows.reshape(BS, SQ, DIM),
                  A.astype(jnp.int32), av, w_3)
    Y, T = _relproj(ego1, trans_M)
    yflat = Y.reshape(N_REL * N_NODE, DIM)
    tflat = T.reshape(N_REL * N_NODE, DIM)

    svec = jnp.zeros((L,), jnp.float32).at[:N_REL].set(jnp.sum(r_embed, axis=1))
    v, mx = _edgescore(yflat, tflat, hl, tl, rl, svec)
    e2, den = _hop2(ego1, hl, tl, v, mx.reshape(NW * 8, L))

    kg = _normkg(ego1, e2.reshape(N_NODE, DIM), den.reshape(N_NODE, DIM))
    idx = inputs.astype(jnp.int32).reshape(-1)
    hrows, kgrows = _assemble(emb, kg, idx)

    av = jnp.concatenate([a_0, a_1, a_2, a_3], axis=1)
    out = _dense(hrows.reshape(BS, SQ, DIM), kgrows.reshape(BS, SQ, DIM),
                 A.astype(jnp.int32), av, w_3)
    return out


# speedup vs baseline: 4.3318x; 1.1287x over previous
import functools

import jax
import jax.numpy as jnp
from jax import lax
from jax.experimental import pallas as pl
from jax.experimental.pallas import tpu as pltpu
from jax.experimental.pallas import tpu_sc as plsc

N_NODE = 10000
N_REL = 8
E = 160000
DIM = 128
BS = 512
SQ = 20
ALPHA = 0.2

NC = 2
NS = 16
NW = NC * NS
L = 16

CH = 128
NCHUNK = E // CH
CPT = (NCHUNK + NW - 1) // NW
CPT1 = (NCHUNK + NS - 1) // NS
NKB = DIM // L

HALF = N_NODE // NC
ACC_ROWS = 5120
RCH = 40
NRCH = HALF // RCH
NZCH = ACC_ROWS // RCH
RCPT = NZCH // NS

_mesh = plsc.VectorSubcoreMesh(core_axis_name="c", subcore_axis_name="s")


def _worker_ids():
    c = lax.axis_index("c")
    s = lax.axis_index("s")
    return c, s, c * NS + s


def _lane_rotsum(x, lanes):
    for sh in (1, 2, 4, 8):
        idx = jnp.bitwise_and(lanes + sh, L - 1)
        x = x + x[idx]
    return x


def _lane_rotmax(x, lanes):
    for sh in (1, 2, 4, 8):
        idx = jnp.bitwise_and(lanes + sh, L - 1)
        x = jnp.maximum(x, x[idx])
    return x


def _zero_vmem(ref, nrows):
    def body(i, _):
        for k in range(ref.shape[1] // L):
            ref[i, pl.ds(k * L, L)] = jnp.zeros((L,), jnp.float32)
        return 0
    lax.fori_loop(0, nrows, body, 0)


NTRI = (CPT1 + 3) // 3


@functools.partial(
    pl.kernel,
    out_type=jax.ShapeDtypeStruct((NC, HALF, DIM), jnp.float32),
    mesh=_mesh,
    scratch_types=[
        [pltpu.VMEM((CH,), jnp.int32) for _ in range(3)],
        [pltpu.VMEM((CH,), jnp.int32) for _ in range(3)],
        pltpu.VMEM((CH,), jnp.float32),
        [pltpu.VMEM((CH, DIM), jnp.float32) for _ in range(3)],
        pltpu.VMEM((RCH, DIM), jnp.float32),
        pltpu.VMEM_SHARED((ACC_ROWS, DIM), jnp.float32),
        [pltpu.SemaphoreType.DMA for _ in range(3)],
        [pltpu.SemaphoreType.DMA for _ in range(3)],
    ],
)
def _spmm1(emb_hbm, hl_hbm, tl_hbm, av_hbm, out_hbm,
           tb, hb, vals, rb, zbuf, acc, sg, ss):
    c, s, w = _worker_ids()
    _zero_vmem(zbuf, RCH)

    def zero(k, _):
        rc = s + NS * k
        pltpu.sync_copy(zbuf, acc.at[pl.ds(rc * RCH, RCH)])
        return 0
    lax.fori_loop(0, RCPT, zero, 0)
    plsc.subcore_barrier()
    lo = c * HALF
    dump = HALF + s

    def load_idx(j, b):
        base = (s + NS * j) * CH
        pltpu.sync_copy(tl_hbm.at[pl.ds(base, CH)], tb[b])
        pltpu.sync_copy(hl_hbm.at[pl.ds(base, CH)], hb[b])
        for k in range(CH // L):
            hloc = hb[b][pl.ds(k * L, L)] - lo
            ok = (hloc >= 0) & (hloc < HALF)
            hb[b][pl.ds(k * L, L)] = jnp.where(ok, hloc, dump)

    load_idx(0, 0)
    pltpu.async_copy(emb_hbm.at[tb[0]], rb[0], sg[0])

    def triple(j3, _):
        for b3 in range(3):
            j = j3 * 3 + b3
            cid = s + NS * j
            nb = (b3 + 1) % 3

            @pl.when(s + NS * (j + 1) < NCHUNK)
            def _():
                def prefetch():
                    load_idx(j + 1, nb)
                    pltpu.async_copy(emb_hbm.at[tb[nb]], rb[nb], sg[nb])
                if b3 == 2:
                    pltpu.make_async_copy(rb[nb], acc.at[hb[nb]], ss[nb]).wait()
                    prefetch()
                else:
                    @pl.when(j3 >= 1)
                    def _():
                        pltpu.make_async_copy(rb[nb], acc.at[hb[nb]], ss[nb]).wait()
                        prefetch()

                    @pl.when(j3 < 1)
                    def _():
                        prefetch()

            @pl.when(cid < NCHUNK)
            def _():
                base = cid * CH
                pltpu.sync_copy(av_hbm.at[pl.ds(base, CH)], vals)
                pltpu.make_async_copy(emb_hbm.at[tb[b3]], rb[b3], sg[b3]).wait()

                @plsc.parallel_loop(0, CH // L, unroll=2)
                def scale(g):
                    v16 = vals[pl.ds(g * L, L)]
                    for e16 in range(L):
                        v = v16[e16]
                        e = g * L + e16
                        for k in range(NKB):
                            rb[b3][e, pl.ds(k * L, L)] = rb[b3][e, pl.ds(k * L, L)] * v
                pltpu.async_copy(rb[b3], acc.at[hb[b3]], ss[b3], add=True)
        return 0
    lax.fori_loop(0, NTRI, triple, 0)
    for b in range(3):
        pltpu.make_async_copy(rb[b], acc.at[hb[b]], ss[b]).wait()
    plsc.subcore_barrier()

    def wb(k, _):
        rc = s + NS * k

        @pl.when(rc < NRCH)
        def _():
            r0 = rc * RCH
            pltpu.sync_copy(acc.at[pl.ds(r0, RCH)], zbuf)
            pltpu.sync_copy(zbuf, out_hbm.at[c, pl.ds(r0, RCH)])
        return 0
    lax.fori_loop(0, RCPT, wb, 0)


def _relproj_body(ego_ref, w_ref, y_ref, t_ref):
    ego = ego_ref[...]
    for r in range(N_REL):
        yr = jnp.dot(ego, w_ref[r], preferred_element_type=jnp.float32)
        y_ref[r] = yr
        t_ref[r] = jnp.tanh(yr)


def _relproj(ego1, trans_M):
    M = 1000
    grid = (N_NODE // M,)
    return pl.pallas_call(
        _relproj_body,
        grid=grid,
        in_specs=[
            pl.BlockSpec((M, DIM), lambda i: (i, 0)),
            pl.BlockSpec((N_REL, DIM, DIM), lambda i: (0, 0, 0)),
        ],
        out_specs=[
            pl.BlockSpec((N_REL, M, DIM), lambda i: (0, i, 0)),
            pl.BlockSpec((N_REL, M, DIM), lambda i: (0, i, 0)),
        ],
        out_shape=[
            jax.ShapeDtypeStruct((N_REL, N_NODE, DIM), jnp.float32),
            jax.ShapeDtypeStruct((N_REL, N_NODE, DIM), jnp.float32),
        ],
    )(ego1, trans_M)


@functools.partial(
    pl.kernel,
    out_type=[
        jax.ShapeDtypeStruct((E,), jnp.float32),
        jax.ShapeDtypeStruct((NW, 8, L), jnp.float32),
    ],
    mesh=_mesh,
    scratch_types=[
        pltpu.VMEM((CH,), jnp.int32),
        pltpu.VMEM((CH,), jnp.int32),
        pltpu.VMEM((CH,), jnp.int32),
        pltpu.VMEM((CH,), jnp.int32),
        pltpu.VMEM((CH,), jnp.int32),
        pltpu.VMEM((CH, DIM), jnp.float32),
        pltpu.VMEM((CH, DIM), jnp.float32),
        pltpu.VMEM((L,), jnp.float32),
        pltpu.VMEM((CH,), jnp.float32),
        pltpu.VMEM((8, L), jnp.float32),
        pltpu.SemaphoreType.DMA,
        pltpu.SemaphoreType.DMA,
    ],
)
def _edgescore(y_hbm, t_hbm, hl_hbm, tl_hbm, rl_hbm, sv_hbm,
               v_hbm, mx_hbm,
               hidx, tidx, ridx, hgat, tgat, yrows, trows, svv, vbuf,
               mrow, sem1, sem2):
    c, s, w = _worker_ids()
    pltpu.sync_copy(sv_hbm, svv)
    lanes = jnp.arange(L, dtype=jnp.int32)
    mrow[0, pl.ds(0, L)] = jnp.full((L,), -3e38, jnp.float32)

    def chunk(j, _):
        cid = w + NW * j

        @pl.when(cid < NCHUNK)
        def do():
            base = cid * CH
            pltpu.sync_copy(hl_hbm.at[pl.ds(base, CH)], hidx)
            pltpu.sync_copy(tl_hbm.at[pl.ds(base, CH)], tidx)
            pltpu.sync_copy(rl_hbm.at[pl.ds(base, CH)], ridx)
            for k in range(CH // L):
                rr = ridx[pl.ds(k * L, L)] * N_NODE
                hgat[pl.ds(k * L, L)] = rr + hidx[pl.ds(k * L, L)]
                tgat[pl.ds(k * L, L)] = rr + tidx[pl.ds(k * L, L)]
            cp1 = pltpu.async_copy(y_hbm.at[hgat], yrows, sem1)
            cp2 = pltpu.async_copy(t_hbm.at[tgat], trows, sem2)
            cp1.wait()
            cp2.wait()

            def group(g, m):
                vvec = jnp.zeros((L,), jnp.float32)
                for e16 in range(L):
                    e = g * L + e16
                    acc = trows[e, pl.ds(0, L)] * yrows[e, pl.ds(0, L)]
                    for k in range(1, NKB):
                        acc = acc + trows[e, pl.ds(k * L, L)] * yrows[e, pl.ds(k * L, L)]
                    accs = _lane_rotsum(acc, lanes)
                    vvec = jnp.where(lanes == e16, accs, vvec)
                r16 = ridx[pl.ds(g * L, L)]
                svvec = svv[pl.ds(0, L)]
                sv = jnp.zeros((L,), jnp.float32)
                for r in range(N_REL):
                    sv = jnp.where(r16 == r, svvec[r], sv)
                vvec = vvec + sv
                vbuf[pl.ds(g * L, L)] = vvec
                return jnp.maximum(m, vvec)
            m0 = mrow[0, pl.ds(0, L)]
            m = plsc.parallel_loop(0, CH // L, unroll=2, carry=m0)(group)
            mrow[0, pl.ds(0, L)] = m
            pltpu.sync_copy(vbuf, v_hbm.at[pl.ds(base, CH)])
        return 0
    lax.fori_loop(0, CPT, chunk, 0)
    m = mrow[0, pl.ds(0, L)]
    for i in range(8):
        mrow[i, pl.ds(0, L)] = m
    pltpu.sync_copy(mrow, mx_hbm.at[w])


def _expv_body(v_ref, mx_ref, ex_ref):
    m = jnp.max(mx_ref[...])
    ex_ref[...] = jnp.exp(v_ref[...] - m)


def _expv(v2d, mx):
    return pl.pallas_call(
        _expv_body,
        grid=(1,),
        in_specs=[
            pl.BlockSpec((E // DIM, DIM), lambda i: (0, 0)),
            pl.BlockSpec((NW * 8, L), lambda i: (0, 0)),
        ],
        out_specs=pl.BlockSpec((E // DIM, DIM), lambda i: (0, 0)),
        out_shape=jax.ShapeDtypeStruct((E // DIM, DIM), jnp.float32),
    )(v2d, mx)


@functools.partial(
    pl.kernel,
    out_type=[
        jax.ShapeDtypeStruct((NC, HALF, DIM), jnp.float32),
        jax.ShapeDtypeStruct((NC, HALF, DIM), jnp.float32),
    ],
    mesh=_mesh,
    scratch_types=[
        pltpu.VMEM((CH,), jnp.int32),
        pltpu.VMEM((CH,), jnp.int32),
        pltpu.VMEM((CH,), jnp.float32),
        pltpu.VMEM((CH, DIM), jnp.float32),
        pltpu.VMEM((CH, DIM), jnp.float32),
        pltpu.VMEM((RCH, DIM), jnp.float32),
        pltpu.VMEM((RCH, DIM), jnp.float32),
        pltpu.VMEM_SHARED((ACC_ROWS, DIM), jnp.float32),
        pltpu.VMEM_SHARED((ACC_ROWS, DIM), jnp.float32),
        pltpu.SemaphoreType.DMA,
    ],
)
def _hop2(ego1_hbm, hl_hbm, tl_hbm, ex_hbm,
          e2_hbm, den_hbm,
          hidx, tidx, vbuf, rows, denrows, zbuf, zbufd, acc, dacc, sem):
    c, s, w = _worker_ids()
    _zero_vmem(zbuf, RCH)
    _zero_vmem(zbufd, RCH)
    _zero_vmem(denrows, CH)

    def zero(k, _):
        rc = s + NS * k
        pltpu.sync_copy(zbuf, acc.at[pl.ds(rc * RCH, RCH)])
        pltpu.sync_copy(zbufd, dacc.at[pl.ds(rc * RCH, RCH)])
        return 0
    lax.fori_loop(0, RCPT, zero, 0)
    plsc.subcore_barrier()
    lo = c * HALF
    dump = HALF + s

    def chunk(j, _):
        cid = s + NS * j

        @pl.when(cid < NCHUNK)
        def _():
            base = cid * CH
            pltpu.sync_copy(ex_hbm.at[pl.ds(base, CH)], vbuf)
            pltpu.sync_copy(hl_hbm.at[pl.ds(base, CH)], hidx)
            pltpu.sync_copy(tl_hbm.at[pl.ds(base, CH)], tidx)
            for k in range(CH // L):
                hloc = hidx[pl.ds(k * L, L)] - lo
                ok = (hloc >= 0) & (hloc < HALF)
                hidx[pl.ds(k * L, L)] = jnp.where(ok, hloc, dump)
            pltpu.async_copy(ego1_hbm.at[tidx], rows, sem).wait()

            @plsc.parallel_loop(0, CH // L, unroll=2)
            def scale(g):
                x16 = vbuf[pl.ds(g * L, L)]
                for e16 in range(L):
                    x = x16[e16]
                    e = g * L + e16
                    xv = jnp.full((L,), x, jnp.float32)
                    denrows[e, pl.ds(0, L)] = xv
                    for k in range(NKB):
                        rows[e, pl.ds(k * L, L)] = rows[e, pl.ds(k * L, L)] * xv
            pltpu.sync_copy(rows, acc.at[hidx], add=True)
            pltpu.sync_copy(denrows, dacc.at[hidx], add=True)
        return 0
    lax.fori_loop(0, CPT1, chunk, 0)
    plsc.subcore_barrier()

    def wb(k, _):
        rc = s + NS * k

        @pl.when(rc < NRCH)
        def _():
            r0 = rc * RCH
            pltpu.sync_copy(acc.at[pl.ds(r0, RCH)], zbuf)
            pltpu.sync_copy(zbuf, e2_hbm.at[c, pl.ds(r0, RCH)])
            pltpu.sync_copy(dacc.at[pl.ds(r0, RCH)], zbufd)
            pltpu.sync_copy(zbufd, den_hbm.at[c, pl.ds(r0, RCH)])
        return 0
    lax.fori_loop(0, RCPT, wb, 0)


def _normkg_body(ego_ref, e2_ref, den_ref, kg_ref):
    d = den_ref[...][:, 0:1]
    d = jnp.where(d == 0.0, 1.0, d)
    kg_ref[...] = (ego_ref[...] + e2_ref[...] / d) * 0.5


def _normkg(ego1, e2, den):
    M = 1000
    return pl.pallas_call(
        _normkg_body,
        grid=(N_NODE // M,),
        in_specs=[
            pl.BlockSpec((M, DIM), lambda i: (i, 0)),
            pl.BlockSpec((M, DIM), lambda i: (i, 0)),
            pl.BlockSpec((M, DIM), lambda i: (i, 0)),
        ],
        out_specs=pl.BlockSpec((M, DIM), lambda i: (i, 0)),
        out_shape=jax.ShapeDtypeStruct((N_NODE, DIM), jnp.float32),
    )(ego1, e2, den)


NIDX = BS * SQ
IPT = NIDX // NW
ICH = 64


@functools.partial(
    pl.kernel,
    out_type=[
        jax.ShapeDtypeStruct((NIDX, DIM), jnp.float32),
        jax.ShapeDtypeStruct((NIDX, DIM), jnp.float32),
    ],
    mesh=_mesh,
    scratch_types=[
        pltpu.VMEM((ICH,), jnp.int32),
        pltpu.VMEM((ICH, DIM), jnp.float32),
        pltpu.VMEM((ICH, DIM), jnp.float32),
        pltpu.SemaphoreType.DMA,
        pltpu.SemaphoreType.DMA,
    ],
)
def _assemble(emb_hbm, kg_hbm,
              idx_hbm, hout_hbm, kgout_hbm,
              idxv, erows, grows, sem1, sem2):
    c, s, w = _worker_ids()

    def chunk(i, _):
        base = w * IPT + i * ICH
        pltpu.sync_copy(idx_hbm.at[pl.ds(base, ICH)], idxv)
        cp1 = pltpu.async_copy(emb_hbm.at[idxv], erows, sem1)
        cp2 = pltpu.async_copy(kg_hbm.at[idxv], grows, sem2)
        cp1.wait()
        cp2.wait()
        pltpu.sync_copy(erows, hout_hbm.at[pl.ds(base, ICH)])
        pltpu.sync_copy(grows, kgout_hbm.at[pl.ds(base, ICH)])
        return 0
    lax.fori_loop(0, IPT // ICH, chunk, 0)


BBLK = 8


def _dense_body(h_ref, kg_ref, a_ref, av_ref, w3_ref, out_ref):
    h = h_ref[...]
    kg = kg_ref[...]
    Ab = a_ref[...]
    av = av_ref[...]
    w3 = w3_ref[...]

    P = h[:, :, None, :] * h[:, None, :, :]
    P = jnp.where(P > 0, P, ALPHA * P)
    e4 = jnp.dot(P.reshape(BBLK * SQ * SQ, DIM), av,
                 preferred_element_type=jnp.float32)
    e4 = e4.reshape(BBLK, SQ, SQ, 4)

    big = jnp.float32(-9e15)
    al = jnp.full((BBLK, SQ, SQ), big, jnp.float32)
    for k in range(4):
        al = jnp.where(Ab == (k + 1), e4[..., k], al)
    al = al - jnp.max(al, axis=-1, keepdims=True)
    al = jnp.exp(al)
    al = al / jnp.sum(al, axis=-1, keepdims=True)

    hid = []
    for b in range(BBLK):
        hid.append(jnp.dot(al[b], h[b], preferred_element_type=jnp.float32))
    hidden = jnp.stack(hid)

    out = jnp.dot(hidden.reshape(BBLK * SQ, DIM), w3[:DIM],
                  preferred_element_type=jnp.float32)
    out = out + jnp.dot(kg.reshape(BBLK * SQ, DIM), w3[DIM:],
                        preferred_element_type=jnp.float32)
    out_ref[...] = jnp.tanh(out).reshape(BBLK, SQ, DIM)


def _dense(hrows, kgrows, A, av, w3):
    grid = (BS // BBLK,)
    return pl.pallas_call(
        _dense_body,
        grid=grid,
        in_specs=[
            pl.BlockSpec((BBLK, SQ, DIM), lambda i: (i, 0, 0)),
            pl.BlockSpec((BBLK, SQ, DIM), lambda i: (i, 0, 0)),
            pl.BlockSpec((BBLK, SQ, SQ), lambda i: (i, 0, 0)),
            pl.BlockSpec((DIM, 4), lambda i: (0, 0)),
            pl.BlockSpec((2 * DIM, DIM), lambda i: (0, 0)),
        ],
        out_specs=pl.BlockSpec((BBLK, SQ, DIM), lambda i: (i, 0, 0)),
        out_shape=jax.ShapeDtypeStruct((BS, SQ, DIM), jnp.float32),
    )(hrows, kgrows, A, av, w3)


def _xla_rest(inputs, A, emb, r_embed, trans_M, hl, tl, rl, A_vals,
              a_0, a_1, a_2, a_3, w_3, ego1, v=None, mx=None, kg=None):
    if v is None:
        Y = jnp.einsum('nd,rde->rne', ego1, trans_M)
        T = jnp.tanh(Y)
        sv = jnp.sum(r_embed, axis=1)
        yf = Y.reshape(N_REL * N_NODE, DIM)
        tf = T.reshape(N_REL * N_NODE, DIM)
        v = jnp.sum(tf[rl * N_NODE + tl] * yf[rl * N_NODE + hl], axis=1) + sv[rl]
    if kg is None:
        gm = jnp.max(v) if mx is None else jnp.max(mx)
        ex = jnp.exp(v - gm)
        den = jax.ops.segment_sum(ex, hl, num_segments=N_NODE)
        e2num = jax.ops.segment_sum(ex[:, None] * ego1[tl], hl, num_segments=N_NODE)
        dv = jnp.where(den == 0, 1., den)
        kg = (ego1 + e2num / dv[:, None]) * 0.5
    idx = inputs.astype(jnp.int32).reshape(-1)
    h = emb[idx].reshape(BS, SQ, DIM)
    kgr = kg[idx].reshape(BS, SQ, DIM)
    ai = h[:, :, None, :] * h[:, None, :, :]
    ai = jnp.where(ai > 0, ai, ALPHA * ai)
    av4 = jnp.concatenate([a_0, a_1, a_2, a_3], axis=1)
    e4 = (ai.reshape(-1, DIM) @ av4).reshape(BS, SQ, SQ, 4)
    al = jnp.full((BS, SQ, SQ), -9e15)
    for k in range(4):
        al = jnp.where(A == k + 1, e4[..., k], al)
    al = jax.nn.softmax(al, axis=-1)
    hid = jnp.einsum('bij,bjd->bid', al, h)
    return jnp.tanh(jnp.concatenate([hid, kgr], axis=-1) @ w_3)


def kernel(inputs, A, mask_item, embedding, r_embed, trans_M, h_list,
           t_list, r_list, A_vals, a_0, a_1, a_2, a_3, w_3):
    hl = h_list.astype(jnp.int32)
    tl = t_list.astype(jnp.int32)
    rl = r_list.astype(jnp.int32)
    emb = embedding.astype(jnp.float32)

    ego1 = _spmm1(emb, hl, tl, A_vals).reshape(N_NODE, DIM)
    Y, T = _relproj(ego1, trans_M)
    yflat = Y.reshape(N_REL * N_NODE, DIM)
    tflat = T.reshape(N_REL * N_NODE, DIM)
    svec = jnp.zeros((L,), jnp.float32).at[:N_REL].set(jnp.sum(r_embed, axis=1))
    v, mx = _edgescore(yflat, tflat, hl, tl, rl, svec)
    ex = _expv(v.reshape(E // DIM, DIM), mx.reshape(NW * 8, L)).reshape(E)
    e2, den = _hop2(ego1, hl, tl, ex)
    kg = _normkg(ego1, e2.reshape(N_NODE, DIM), den.reshape(N_NODE, DIM))
    idx = inputs.astype(jnp.int32).reshape(-1)
    hrows, kgrows = _assemble(emb, kg, idx)
    av = jnp.concatenate([a_0, a_1, a_2, a_3], axis=1)
    return _dense(hrows.reshape(BS, SQ, DIM), kgrows.reshape(BS, SQ, DIM),
                  A.astype(jnp.int32), av, w_3)
    Y, T = _relproj(ego1, trans_M)
    yflat = Y.reshape(N_REL * N_NODE, DIM)
    tflat = T.reshape(N_REL * N_NODE, DIM)

    svec = jnp.zeros((L,), jnp.float32).at[:N_REL].set(jnp.sum(r_embed, axis=1))
    v, mx = _edgescore(yflat, tflat, hl, tl, rl, svec)
    e2, den = _hop2(ego1, hl, tl, v, mx.reshape(NW * 8, L))

    kg = _normkg(ego1, e2.reshape(N_NODE, DIM), den.reshape(N_NODE, DIM))
    idx = inputs.astype(jnp.int32).reshape(-1)
    hrows, kgrows = _assemble(emb, kg, idx)

    av = jnp.concatenate([a_0, a_1, a_2, a_3], axis=1)
    out = _dense(hrows.reshape(BS, SQ, DIM), kgrows.reshape(BS, SQ, DIM),
                 A.astype(jnp.int32), av, w_3)
    return out


# speedup vs baseline: 4.3363x; 1.0010x over previous
import functools

import jax
import jax.numpy as jnp
from jax import lax
from jax.experimental import pallas as pl
from jax.experimental.pallas import tpu as pltpu
from jax.experimental.pallas import tpu_sc as plsc

N_NODE = 10000
N_REL = 8
E = 160000
DIM = 128
BS = 512
SQ = 20
ALPHA = 0.2

NC = 2
NS = 16
NW = NC * NS
L = 16

CH = 128
NCHUNK = E // CH
CPT = (NCHUNK + NW - 1) // NW
CPT1 = (NCHUNK + NS - 1) // NS
NKB = DIM // L

HALF = N_NODE // NC
ACC_ROWS = 5120
RCH = 40
NRCH = HALF // RCH
NZCH = ACC_ROWS // RCH
RCPT = NZCH // NS

_mesh = plsc.VectorSubcoreMesh(core_axis_name="c", subcore_axis_name="s")


def _worker_ids():
    c = lax.axis_index("c")
    s = lax.axis_index("s")
    return c, s, c * NS + s


def _lane_rotsum(x, lanes):
    for sh in (1, 2, 4, 8):
        idx = jnp.bitwise_and(lanes + sh, L - 1)
        x = x + x[idx]
    return x


def _lane_rotmax(x, lanes):
    for sh in (1, 2, 4, 8):
        idx = jnp.bitwise_and(lanes + sh, L - 1)
        x = jnp.maximum(x, x[idx])
    return x


def _zero_vmem(ref, nrows):
    def body(i, _):
        for k in range(ref.shape[1] // L):
            ref[i, pl.ds(k * L, L)] = jnp.zeros((L,), jnp.float32)
        return 0
    lax.fori_loop(0, nrows, body, 0)


NTRI = (CPT1 + 3) // 3


@functools.partial(
    pl.kernel,
    out_type=jax.ShapeDtypeStruct((NC, HALF, DIM), jnp.float32),
    mesh=_mesh,
    scratch_types=[
        [pltpu.VMEM((CH,), jnp.int32) for _ in range(3)],
        [pltpu.VMEM((CH,), jnp.int32) for _ in range(3)],
        pltpu.VMEM((CH,), jnp.float32),
        [pltpu.VMEM((CH, DIM), jnp.float32) for _ in range(3)],
        pltpu.VMEM((RCH, DIM), jnp.float32),
        pltpu.VMEM_SHARED((ACC_ROWS, DIM), jnp.float32),
        [pltpu.SemaphoreType.DMA for _ in range(3)],
        [pltpu.SemaphoreType.DMA for _ in range(3)],
    ],
)
def _spmm1(emb_hbm, hl_hbm, tl_hbm, av_hbm, out_hbm,
           tb, hb, vals, rb, zbuf, acc, sg, ss):
    c, s, w = _worker_ids()
    _zero_vmem(zbuf, RCH)

    def zero(k, _):
        rc = s + NS * k
        pltpu.sync_copy(zbuf, acc.at[pl.ds(rc * RCH, RCH)])
        return 0
    lax.fori_loop(0, RCPT, zero, 0)
    plsc.subcore_barrier()
    lo = c * HALF
    dump = HALF + s

    def load_idx(j, b):
        base = (s + NS * j) * CH
        pltpu.sync_copy(tl_hbm.at[pl.ds(base, CH)], tb[b])
        pltpu.sync_copy(hl_hbm.at[pl.ds(base, CH)], hb[b])
        for k in range(CH // L):
            hloc = hb[b][pl.ds(k * L, L)] - lo
            ok = (hloc >= 0) & (hloc < HALF)
            hb[b][pl.ds(k * L, L)] = jnp.where(ok, hloc, dump)

    load_idx(0, 0)
    pltpu.async_copy(emb_hbm.at[tb[0]], rb[0], sg[0])

    def triple(j3, _):
        for b3 in range(3):
            j = j3 * 3 + b3
            cid = s + NS * j
            nb = (b3 + 1) % 3

            @pl.when(s + NS * (j + 1) < NCHUNK)
            def _():
                def prefetch():
                    load_idx(j + 1, nb)
                    pltpu.async_copy(emb_hbm.at[tb[nb]], rb[nb], sg[nb])
                if b3 == 2:
                    pltpu.make_async_copy(rb[nb], acc.at[hb[nb]], ss[nb]).wait()
                    prefetch()
                else:
                    @pl.when(j3 >= 1)
                    def _():
                        pltpu.make_async_copy(rb[nb], acc.at[hb[nb]], ss[nb]).wait()
                        prefetch()

                    @pl.when(j3 < 1)
                    def _():
                        prefetch()

            @pl.when(cid < NCHUNK)
            def _():
                base = cid * CH
                pltpu.sync_copy(av_hbm.at[pl.ds(base, CH)], vals)
                pltpu.make_async_copy(emb_hbm.at[tb[b3]], rb[b3], sg[b3]).wait()

                @plsc.parallel_loop(0, CH // L, unroll=2)
                def scale(g):
                    v16 = vals[pl.ds(g * L, L)]
                    for e16 in range(L):
                        v = v16[e16]
                        e = g * L + e16
                        for k in range(NKB):
                            rb[b3][e, pl.ds(k * L, L)] = rb[b3][e, pl.ds(k * L, L)] * v
                pltpu.async_copy(rb[b3], acc.at[hb[b3]], ss[b3], add=True)
        return 0
    lax.fori_loop(0, NTRI, triple, 0)
    for b in range(3):
        pltpu.make_async_copy(rb[b], acc.at[hb[b]], ss[b]).wait()
    plsc.subcore_barrier()

    def wb(k, _):
        rc = s + NS * k

        @pl.when(rc < NRCH)
        def _():
            r0 = rc * RCH
            pltpu.sync_copy(acc.at[pl.ds(r0, RCH)], zbuf)
            pltpu.sync_copy(zbuf, out_hbm.at[c, pl.ds(r0, RCH)])
        return 0
    lax.fori_loop(0, RCPT, wb, 0)


def _relproj_body(ego_ref, w_ref, y_ref, t_ref):
    ego = ego_ref[...]
    for r in range(N_REL):
        yr = jnp.dot(ego, w_ref[r], preferred_element_type=jnp.float32)
        y_ref[r] = yr
        t_ref[r] = jnp.tanh(yr)


def _relproj(ego1, trans_M):
    M = 1000
    grid = (N_NODE // M,)
    return pl.pallas_call(
        _relproj_body,
        grid=grid,
        in_specs=[
            pl.BlockSpec((M, DIM), lambda i: (i, 0)),
            pl.BlockSpec((N_REL, DIM, DIM), lambda i: (0, 0, 0)),
        ],
        out_specs=[
            pl.BlockSpec((N_REL, M, DIM), lambda i: (0, i, 0)),
            pl.BlockSpec((N_REL, M, DIM), lambda i: (0, i, 0)),
        ],
        out_shape=[
            jax.ShapeDtypeStruct((N_REL, N_NODE, DIM), jnp.float32),
            jax.ShapeDtypeStruct((N_REL, N_NODE, DIM), jnp.float32),
        ],
    )(ego1, trans_M)


@functools.partial(
    pl.kernel,
    out_type=[
        jax.ShapeDtypeStruct((E,), jnp.float32),
        jax.ShapeDtypeStruct((NW, 8, L), jnp.float32),
    ],
    mesh=_mesh,
    scratch_types=[
        pltpu.VMEM((CH,), jnp.int32),
        pltpu.VMEM((CH,), jnp.int32),
        pltpu.VMEM((CH,), jnp.int32),
        pltpu.VMEM((CH,), jnp.int32),
        pltpu.VMEM((CH,), jnp.int32),
        pltpu.VMEM((CH, DIM), jnp.float32),
        pltpu.VMEM((CH, DIM), jnp.float32),
        pltpu.VMEM((L,), jnp.float32),
        pltpu.VMEM((CH,), jnp.float32),
        pltpu.VMEM((8, L), jnp.float32),
        pltpu.SemaphoreType.DMA,
        pltpu.SemaphoreType.DMA,
    ],
)
def _edgescore(y_hbm, t_hbm, hl_hbm, tl_hbm, rl_hbm, sv_hbm,
               v_hbm, mx_hbm,
               hidx, tidx, ridx, hgat, tgat, yrows, trows, svv, vbuf,
               mrow, sem1, sem2):
    c, s, w = _worker_ids()
    pltpu.sync_copy(sv_hbm, svv)
    lanes = jnp.arange(L, dtype=jnp.int32)
    mrow[0, pl.ds(0, L)] = jnp.full((L,), -3e38, jnp.float32)

    def chunk(j, _):
        cid = w + NW * j

        @pl.when(cid < NCHUNK)
        def do():
            base = cid * CH
            pltpu.sync_copy(hl_hbm.at[pl.ds(base, CH)], hidx)
            pltpu.sync_copy(tl_hbm.at[pl.ds(base, CH)], tidx)
            pltpu.sync_copy(rl_hbm.at[pl.ds(base, CH)], ridx)
            for k in range(CH // L):
                rr = ridx[pl.ds(k * L, L)] * N_NODE
                hgat[pl.ds(k * L, L)] = rr + hidx[pl.ds(k * L, L)]
                tgat[pl.ds(k * L, L)] = rr + tidx[pl.ds(k * L, L)]
            cp1 = pltpu.async_copy(y_hbm.at[hgat], yrows, sem1)
            cp2 = pltpu.async_copy(t_hbm.at[tgat], trows, sem2)
            cp1.wait()
            cp2.wait()

            def group(g, m):
                vvec = jnp.zeros((L,), jnp.float32)
                for e16 in range(L):
                    e = g * L + e16
                    acc = trows[e, pl.ds(0, L)] * yrows[e, pl.ds(0, L)]
                    for k in range(1, NKB):
                        acc = acc + trows[e, pl.ds(k * L, L)] * yrows[e, pl.ds(k * L, L)]
                    accs = _lane_rotsum(acc, lanes)
                    vvec = jnp.where(lanes == e16, accs, vvec)
                r16 = ridx[pl.ds(g * L, L)]
                svvec = svv[pl.ds(0, L)]
                sv = jnp.zeros((L,), jnp.float32)
                for r in range(N_REL):
                    sv = jnp.where(r16 == r, svvec[r], sv)
                vvec = vvec + sv
                vbuf[pl.ds(g * L, L)] = vvec
                return jnp.maximum(m, vvec)
            m0 = mrow[0, pl.ds(0, L)]
            m = plsc.parallel_loop(0, CH // L, unroll=2, carry=m0)(group)
            mrow[0, pl.ds(0, L)] = m
            pltpu.sync_copy(vbuf, v_hbm.at[pl.ds(base, CH)])
        return 0
    lax.fori_loop(0, CPT, chunk, 0)
    m = mrow[0, pl.ds(0, L)]
    for i in range(8):
        mrow[i, pl.ds(0, L)] = m
    pltpu.sync_copy(mrow, mx_hbm.at[w])


def _expv_body(v_ref, mx_ref, ex_ref):
    m = jnp.max(mx_ref[...])
    ex_ref[...] = jnp.exp(v_ref[...] - m)


def _expv(v2d, mx):
    return pl.pallas_call(
        _expv_body,
        grid=(1,),
        in_specs=[
            pl.BlockSpec((E // DIM, DIM), lambda i: (0, 0)),
            pl.BlockSpec((NW * 8, L), lambda i: (0, 0)),
        ],
        out_specs=pl.BlockSpec((E // DIM, DIM), lambda i: (0, 0)),
        out_shape=jax.ShapeDtypeStruct((E // DIM, DIM), jnp.float32),
    )(v2d, mx)


@functools.partial(
    pl.kernel,
    out_type=[
        jax.ShapeDtypeStruct((NC, HALF, DIM), jnp.float32),
        jax.ShapeDtypeStruct((NC, HALF, DIM), jnp.float32),
    ],
    mesh=_mesh,
    scratch_types=[
        pltpu.VMEM((CH,), jnp.int32),
        pltpu.VMEM((CH,), jnp.int32),
        pltpu.VMEM((CH,), jnp.float32),
        pltpu.VMEM((CH, DIM), jnp.float32),
        pltpu.VMEM((CH, DIM), jnp.float32),
        pltpu.VMEM((RCH, DIM), jnp.float32),
        pltpu.VMEM((RCH, DIM), jnp.float32),
        pltpu.VMEM_SHARED((ACC_ROWS, DIM), jnp.float32),
        pltpu.VMEM_SHARED((ACC_ROWS, DIM), jnp.float32),
        pltpu.SemaphoreType.DMA,
    ],
)
def _hop2(ego1_hbm, hl_hbm, tl_hbm, ex_hbm,
          e2_hbm, den_hbm,
          hidx, tidx, vbuf, rows, db, zbuf, zbufd, acc, dacc, sem):
    c, s, w = _worker_ids()
    _zero_vmem(zbuf, RCH)
    _zero_vmem(zbufd, RCH)
    _zero_vmem(db, CH)

    def zero(k, _):
        rc = s + NS * k
        pltpu.sync_copy(zbuf, acc.at[pl.ds(rc * RCH, RCH)])
        pltpu.sync_copy(zbufd, dacc.at[pl.ds(rc * RCH, RCH)])
        return 0
    lax.fori_loop(0, RCPT, zero, 0)
    plsc.subcore_barrier()
    lo = c * HALF
    dump = HALF + s

    def chunk(j, _):
        cid = s + NS * j

        @pl.when(cid < NCHUNK)
        def _():
            base = cid * CH
            pltpu.sync_copy(ex_hbm.at[pl.ds(base, CH)], vbuf)
            pltpu.sync_copy(hl_hbm.at[pl.ds(base, CH)], hidx)
            pltpu.sync_copy(tl_hbm.at[pl.ds(base, CH)], tidx)
            for k in range(CH // L):
                hloc = hidx[pl.ds(k * L, L)] - lo
                ok = (hloc >= 0) & (hloc < HALF)
                hidx[pl.ds(k * L, L)] = jnp.where(ok, hloc, dump)
            pltpu.async_copy(ego1_hbm.at[tidx], rows, sem).wait()

            @plsc.parallel_loop(0, CH // L, unroll=2)
            def scale(g):
                x16 = vbuf[pl.ds(g * L, L)]
                for e16 in range(L):
                    x = x16[e16]
                    e = g * L + e16
                    xv = jnp.full((L,), x, jnp.float32)
                    db[e, pl.ds(0, L)] = xv
                    for k in range(NKB):
                        rows[e, pl.ds(k * L, L)] = rows[e, pl.ds(k * L, L)] * xv
            pltpu.sync_copy(rows, acc.at[hidx], add=True)
            pltpu.sync_copy(db, dacc.at[hidx], add=True)
        return 0
    lax.fori_loop(0, CPT1, chunk, 0)
    plsc.subcore_barrier()

    def wb(k, _):
        rc = s + NS * k

        @pl.when(rc < NRCH)
        def _():
            r0 = rc * RCH
            pltpu.sync_copy(acc.at[pl.ds(r0, RCH)], zbuf)
            pltpu.sync_copy(zbuf, e2_hbm.at[c, pl.ds(r0, RCH)])
            pltpu.sync_copy(dacc.at[pl.ds(r0, RCH)], zbufd)
            pltpu.sync_copy(zbufd, den_hbm.at[c, pl.ds(r0, RCH)])
        return 0
    lax.fori_loop(0, RCPT, wb, 0)


def _normkg_body(ego_ref, e2_ref, den_ref, kg_ref):
    d = den_ref[...][:, 0:1]
    d = jnp.where(d == 0.0, 1.0, d)
    kg_ref[...] = (ego_ref[...] + e2_ref[...] / d) * 0.5


def _normkg(ego1, e2, den):
    M = 1000
    return pl.pallas_call(
        _normkg_body,
        grid=(N_NODE // M,),
        in_specs=[
            pl.BlockSpec((M, DIM), lambda i: (i, 0)),
            pl.BlockSpec((M, DIM), lambda i: (i, 0)),
            pl.BlockSpec((M, DIM), lambda i: (i, 0)),
        ],
        out_specs=pl.BlockSpec((M, DIM), lambda i: (i, 0)),
        out_shape=jax.ShapeDtypeStruct((N_NODE, DIM), jnp.float32),
    )(ego1, e2, den)


NIDX = BS * SQ
IPT = NIDX // NW
ICH = 64


@functools.partial(
    pl.kernel,
    out_type=[
        jax.ShapeDtypeStruct((NIDX, DIM), jnp.float32),
        jax.ShapeDtypeStruct((NIDX, DIM), jnp.float32),
    ],
    mesh=_mesh,
    scratch_types=[
        pltpu.VMEM((ICH,), jnp.int32),
        pltpu.VMEM((ICH, DIM), jnp.float32),
        pltpu.VMEM((ICH, DIM), jnp.float32),
        pltpu.SemaphoreType.DMA,
        pltpu.SemaphoreType.DMA,
    ],
)
def _assemble(emb_hbm, kg_hbm,
              idx_hbm, hout_hbm, kgout_hbm,
              idxv, erows, grows, sem1, sem2):
    c, s, w = _worker_ids()

    def chunk(i, _):
        base = w * IPT + i * ICH
        pltpu.sync_copy(idx_hbm.at[pl.ds(base, ICH)], idxv)
        cp1 = pltpu.async_copy(emb_hbm.at[idxv], erows, sem1)
        cp2 = pltpu.async_copy(kg_hbm.at[idxv], grows, sem2)
        cp1.wait()
        cp2.wait()
        pltpu.sync_copy(erows, hout_hbm.at[pl.ds(base, ICH)])
        pltpu.sync_copy(grows, kgout_hbm.at[pl.ds(base, ICH)])
        return 0
    lax.fori_loop(0, IPT // ICH, chunk, 0)


BBLK = 8


def _dense_body(h_ref, kg_ref, a_ref, av_ref, w3_ref, out_ref):
    h = h_ref[...]
    kg = kg_ref[...]
    Ab = a_ref[...]
    av = av_ref[...]
    w3 = w3_ref[...]

    P = h[:, :, None, :] * h[:, None, :, :]
    P = jnp.where(P > 0, P, ALPHA * P)
    e4 = jnp.dot(P.reshape(BBLK * SQ * SQ, DIM), av,
                 preferred_element_type=jnp.float32)
    e4 = e4.reshape(BBLK, SQ, SQ, 4)

    big = jnp.float32(-9e15)
    al = jnp.full((BBLK, SQ, SQ), big, jnp.float32)
    for k in range(4):
        al = jnp.where(Ab == (k + 1), e4[..., k], al)
    al = al - jnp.max(al, axis=-1, keepdims=True)
    al = jnp.exp(al)
    al = al / jnp.sum(al, axis=-1, keepdims=True)

    hid = []
    for b in range(BBLK):
        hid.append(jnp.dot(al[b], h[b], preferred_element_type=jnp.float32))
    hidden = jnp.stack(hid)

    out = jnp.dot(hidden.reshape(BBLK * SQ, DIM), w3[:DIM],
                  preferred_element_type=jnp.float32)
    out = out + jnp.dot(kg.reshape(BBLK * SQ, DIM), w3[DIM:],
                        preferred_element_type=jnp.float32)
    out_ref[...] = jnp.tanh(out).reshape(BBLK, SQ, DIM)


def _dense(hrows, kgrows, A, av, w3):
    grid = (BS // BBLK,)
    return pl.pallas_call(
        _dense_body,
        grid=grid,
        in_specs=[
            pl.BlockSpec((BBLK, SQ, DIM), lambda i: (i, 0, 0)),
            pl.BlockSpec((BBLK, SQ, DIM), lambda i: (i, 0, 0)),
            pl.BlockSpec((BBLK, SQ, SQ), lambda i: (i, 0, 0)),
            pl.BlockSpec((DIM, 4), lambda i: (0, 0)),
            pl.BlockSpec((2 * DIM, DIM), lambda i: (0, 0)),
        ],
        out_specs=pl.BlockSpec((BBLK, SQ, DIM), lambda i: (i, 0, 0)),
        out_shape=jax.ShapeDtypeStruct((BS, SQ, DIM), jnp.float32),
    )(hrows, kgrows, A, av, w3)


def _xla_rest(inputs, A, emb, r_embed, trans_M, hl, tl, rl, A_vals,
              a_0, a_1, a_2, a_3, w_3, ego1, v=None, mx=None, kg=None):
    if v is None:
        Y = jnp.einsum('nd,rde->rne', ego1, trans_M)
        T = jnp.tanh(Y)
        sv = jnp.sum(r_embed, axis=1)
        yf = Y.reshape(N_REL * N_NODE, DIM)
        tf = T.reshape(N_REL * N_NODE, DIM)
        v = jnp.sum(tf[rl * N_NODE + tl] * yf[rl * N_NODE + hl], axis=1) + sv[rl]
    if kg is None:
        gm = jnp.max(v) if mx is None else jnp.max(mx)
        ex = jnp.exp(v - gm)
        den = jax.ops.segment_sum(ex, hl, num_segments=N_NODE)
        e2num = jax.ops.segment_sum(ex[:, None] * ego1[tl], hl, num_segments=N_NODE)
        dv = jnp.where(den == 0, 1., den)
        kg = (ego1 + e2num / dv[:, None]) * 0.5
    idx = inputs.astype(jnp.int32).reshape(-1)
    h = emb[idx].reshape(BS, SQ, DIM)
    kgr = kg[idx].reshape(BS, SQ, DIM)
    ai = h[:, :, None, :] * h[:, None, :, :]
    ai = jnp.where(ai > 0, ai, ALPHA * ai)
    av4 = jnp.concatenate([a_0, a_1, a_2, a_3], axis=1)
    e4 = (ai.reshape(-1, DIM) @ av4).reshape(BS, SQ, SQ, 4)
    al = jnp.full((BS, SQ, SQ), -9e15)
    for k in range(4):
        al = jnp.where(A == k + 1, e4[..., k], al)
    al = jax.nn.softmax(al, axis=-1)
    hid = jnp.einsum('bij,bjd->bid', al, h)
    return jnp.tanh(jnp.concatenate([hid, kgr], axis=-1) @ w_3)


def kernel(inputs, A, mask_item, embedding, r_embed, trans_M, h_list,
           t_list, r_list, A_vals, a_0, a_1, a_2, a_3, w_3):
    hl = h_list.astype(jnp.int32)
    tl = t_list.astype(jnp.int32)
    rl = r_list.astype(jnp.int32)
    emb = embedding.astype(jnp.float32)

    ego1 = _spmm1(emb, hl, tl, A_vals).reshape(N_NODE, DIM)
    Y, T = _relproj(ego1, trans_M)
    yflat = Y.reshape(N_REL * N_NODE, DIM)
    tflat = T.reshape(N_REL * N_NODE, DIM)
    svec = jnp.zeros((L,), jnp.float32).at[:N_REL].set(jnp.sum(r_embed, axis=1))
    v, mx = _edgescore(yflat, tflat, hl, tl, rl, svec)
    ex = _expv(v.reshape(E // DIM, DIM), mx.reshape(NW * 8, L)).reshape(E)
    e2, den = _hop2(ego1, hl, tl, ex)
    kg = _normkg(ego1, e2.reshape(N_NODE, DIM), den.reshape(N_NODE, DIM))
    idx = inputs.astype(jnp.int32).reshape(-1)
    hrows, kgrows = _assemble(emb, kg, idx)
    av = jnp.concatenate([a_0, a_1, a_2, a_3], axis=1)
    return _dense(hrows.reshape(BS, SQ, DIM), kgrows.reshape(BS, SQ, DIM),
                  A.astype(jnp.int32), av, w_3)
    Y, T = _relproj(ego1, trans_M)
    yflat = Y.reshape(N_REL * N_NODE, DIM)
    tflat = T.reshape(N_REL * N_NODE, DIM)

    svec = jnp.zeros((L,), jnp.float32).at[:N_REL].set(jnp.sum(r_embed, axis=1))
    v, mx = _edgescore(yflat, tflat, hl, tl, rl, svec)
    e2, den = _hop2(ego1, hl, tl, v, mx.reshape(NW * 8, L))

    kg = _normkg(ego1, e2.reshape(N_NODE, DIM), den.reshape(N_NODE, DIM))
    idx = inputs.astype(jnp.int32).reshape(-1)
    hrows, kgrows = _assemble(emb, kg, idx)

    av = jnp.concatenate([a_0, a_1, a_2, a_3], axis=1)
    out = _dense(hrows.reshape(BS, SQ, DIM), kgrows.reshape(BS, SQ, DIM),
                 A.astype(jnp.int32), av, w_3)
    return out


# speedup vs baseline: 4.6580x; 1.0742x over previous
import functools

import jax
import jax.numpy as jnp
from jax import lax
from jax.experimental import pallas as pl
from jax.experimental.pallas import tpu as pltpu
from jax.experimental.pallas import tpu_sc as plsc

N_NODE = 10000
N_REL = 8
E = 160000
DIM = 128
BS = 512
SQ = 20
ALPHA = 0.2

NC = 2
NS = 16
NW = NC * NS
L = 16

CH = 128
NCHUNK = E // CH
CPT = (NCHUNK + NW - 1) // NW
CPT1 = (NCHUNK + NS - 1) // NS
NKB = DIM // L

HALF = N_NODE // NC
ACC_ROWS = 5120
RCH = 40
NRCH = HALF // RCH
NZCH = ACC_ROWS // RCH
RCPT = NZCH // NS

_mesh = plsc.VectorSubcoreMesh(core_axis_name="c", subcore_axis_name="s")


def _worker_ids():
    c = lax.axis_index("c")
    s = lax.axis_index("s")
    return c, s, c * NS + s


def _lane_rotsum(x, lanes):
    for sh in (1, 2, 4, 8):
        idx = jnp.bitwise_and(lanes + sh, L - 1)
        x = x + x[idx]
    return x


def _lane_rotmax(x, lanes):
    for sh in (1, 2, 4, 8):
        idx = jnp.bitwise_and(lanes + sh, L - 1)
        x = jnp.maximum(x, x[idx])
    return x


def _zero_vmem(ref, nrows):
    def body(i, _):
        for k in range(ref.shape[1] // L):
            ref[i, pl.ds(k * L, L)] = jnp.zeros((L,), jnp.float32)
        return 0
    lax.fori_loop(0, nrows, body, 0)


NTRI = (CPT1 + 3) // 3


@functools.partial(
    pl.kernel,
    out_type=jax.ShapeDtypeStruct((NC, HALF, DIM), jnp.float32),
    mesh=_mesh,
    scratch_types=[
        [pltpu.VMEM((CH,), jnp.int32) for _ in range(3)],
        [pltpu.VMEM((CH,), jnp.int32) for _ in range(3)],
        pltpu.VMEM((CH,), jnp.float32),
        [pltpu.VMEM((CH, DIM), jnp.float32) for _ in range(3)],
        pltpu.VMEM((RCH, DIM), jnp.float32),
        pltpu.VMEM_SHARED((ACC_ROWS, DIM), jnp.float32),
        [pltpu.SemaphoreType.DMA for _ in range(3)],
        [pltpu.SemaphoreType.DMA for _ in range(3)],
    ],
)
def _spmm1(emb_hbm, hl_hbm, tl_hbm, av_hbm, out_hbm,
           tb, hb, vals, rb, zbuf, acc, sg, ss):
    c, s, w = _worker_ids()
    _zero_vmem(zbuf, RCH)

    def zero(k, _):
        rc = s + NS * k
        pltpu.sync_copy(zbuf, acc.at[pl.ds(rc * RCH, RCH)])
        return 0
    lax.fori_loop(0, RCPT, zero, 0)
    plsc.subcore_barrier()
    lo = c * HALF
    dump = HALF + s

    def load_idx(j, b):
        base = (s + NS * j) * CH
        pltpu.sync_copy(tl_hbm.at[pl.ds(base, CH)], tb[b])
        pltpu.sync_copy(hl_hbm.at[pl.ds(base, CH)], hb[b])
        for k in range(CH // L):
            hloc = hb[b][pl.ds(k * L, L)] - lo
            ok = (hloc >= 0) & (hloc < HALF)
            hb[b][pl.ds(k * L, L)] = jnp.where(ok, hloc, dump)

    load_idx(0, 0)
    pltpu.async_copy(emb_hbm.at[tb[0]], rb[0], sg[0])

    def triple(j3, _):
        for b3 in range(3):
            j = j3 * 3 + b3
            cid = s + NS * j
            nb = (b3 + 1) % 3

            @pl.when(s + NS * (j + 1) < NCHUNK)
            def _():
                def prefetch():
                    load_idx(j + 1, nb)
                    pltpu.async_copy(emb_hbm.at[tb[nb]], rb[nb], sg[nb])
                if b3 == 2:
                    pltpu.make_async_copy(rb[nb], acc.at[hb[nb]], ss[nb]).wait()
                    prefetch()
                else:
                    @pl.when(j3 >= 1)
                    def _():
                        pltpu.make_async_copy(rb[nb], acc.at[hb[nb]], ss[nb]).wait()
                        prefetch()

                    @pl.when(j3 < 1)
                    def _():
                        prefetch()

            @pl.when(cid < NCHUNK)
            def _():
                base = cid * CH
                pltpu.sync_copy(av_hbm.at[pl.ds(base, CH)], vals)
                pltpu.make_async_copy(emb_hbm.at[tb[b3]], rb[b3], sg[b3]).wait()

                @plsc.parallel_loop(0, CH // L, unroll=2)
                def scale(g):
                    v16 = vals[pl.ds(g * L, L)]
                    for e16 in range(L):
                        v = v16[e16]
                        e = g * L + e16
                        for k in range(NKB):
                            rb[b3][e, pl.ds(k * L, L)] = rb[b3][e, pl.ds(k * L, L)] * v
                pltpu.async_copy(rb[b3], acc.at[hb[b3]], ss[b3], add=True)
        return 0
    lax.fori_loop(0, NTRI, triple, 0)
    for b in range(3):
        pltpu.make_async_copy(rb[b], acc.at[hb[b]], ss[b]).wait()
    plsc.subcore_barrier()

    def wb(k, _):
        rc = s + NS * k

        @pl.when(rc < NRCH)
        def _():
            r0 = rc * RCH
            pltpu.sync_copy(acc.at[pl.ds(r0, RCH)], zbuf)
            pltpu.sync_copy(zbuf, out_hbm.at[c, pl.ds(r0, RCH)])
        return 0
    lax.fori_loop(0, RCPT, wb, 0)


def _relproj_body(ego_ref, w_ref, y_ref, t_ref):
    ego = ego_ref[...]
    for r in range(N_REL):
        yr = jnp.dot(ego, w_ref[r], preferred_element_type=jnp.float32)
        y_ref[r] = yr
        t_ref[r] = jnp.tanh(yr)


def _relproj(ego1, trans_M):
    M = 1000
    grid = (N_NODE // M,)
    return pl.pallas_call(
        _relproj_body,
        grid=grid,
        in_specs=[
            pl.BlockSpec((M, DIM), lambda i: (i, 0)),
            pl.BlockSpec((N_REL, DIM, DIM), lambda i: (0, 0, 0)),
        ],
        out_specs=[
            pl.BlockSpec((N_REL, M, DIM), lambda i: (0, i, 0)),
            pl.BlockSpec((N_REL, M, DIM), lambda i: (0, i, 0)),
        ],
        out_shape=[
            jax.ShapeDtypeStruct((N_REL, N_NODE, DIM), jnp.float32),
            jax.ShapeDtypeStruct((N_REL, N_NODE, DIM), jnp.float32),
        ],
    )(ego1, trans_M)


@functools.partial(
    pl.kernel,
    out_type=[
        jax.ShapeDtypeStruct((E,), jnp.float32),
        jax.ShapeDtypeStruct((NW, 8, L), jnp.float32),
    ],
    mesh=_mesh,
    scratch_types=[
        pltpu.VMEM((CH,), jnp.int32),
        pltpu.VMEM((CH,), jnp.int32),
        [pltpu.VMEM((CH,), jnp.int32) for _ in range(2)],
        [pltpu.VMEM((CH,), jnp.int32) for _ in range(2)],
        [pltpu.VMEM((CH,), jnp.int32) for _ in range(2)],
        [pltpu.VMEM((CH, DIM), jnp.float32) for _ in range(2)],
        [pltpu.VMEM((CH, DIM), jnp.float32) for _ in range(2)],
        pltpu.VMEM((L,), jnp.float32),
        pltpu.VMEM((CH,), jnp.float32),
        pltpu.VMEM((8, L), jnp.float32),
        [pltpu.SemaphoreType.DMA for _ in range(2)],
        [pltpu.SemaphoreType.DMA for _ in range(2)],
    ],
)
def _edgescore(y_hbm, t_hbm, hl_hbm, tl_hbm, rl_hbm, sv_hbm,
               v_hbm, mx_hbm,
               hidx, tidx, ridx, hgat, tgat, yrows, trows, svv, vbuf,
               mrow, sgy, sgt):
    c, s, w = _worker_ids()
    pltpu.sync_copy(sv_hbm, svv)
    lanes = jnp.arange(L, dtype=jnp.int32)
    mrow[0, pl.ds(0, L)] = jnp.full((L,), -3e38, jnp.float32)

    def prefetch(j, b):
        base = (w + NW * j) * CH
        pltpu.sync_copy(hl_hbm.at[pl.ds(base, CH)], hidx)
        pltpu.sync_copy(tl_hbm.at[pl.ds(base, CH)], tidx)
        pltpu.sync_copy(rl_hbm.at[pl.ds(base, CH)], ridx[b])
        for k in range(CH // L):
            rr = ridx[b][pl.ds(k * L, L)] * N_NODE
            hgat[b][pl.ds(k * L, L)] = rr + hidx[pl.ds(k * L, L)]
            tgat[b][pl.ds(k * L, L)] = rr + tidx[pl.ds(k * L, L)]
        pltpu.async_copy(y_hbm.at[hgat[b]], yrows[b], sgy[b])
        pltpu.async_copy(t_hbm.at[tgat[b]], trows[b], sgt[b])

    prefetch(0, 0)

    def pair(j2, _):
        for b in range(2):
            j = j2 * 2 + b
            cid = w + NW * j
            nb = 1 - b

            @pl.when(w + NW * (j + 1) < NCHUNK)
            def _():
                prefetch(j + 1, nb)

            @pl.when(cid < NCHUNK)
            def do():
                base = cid * CH
                pltpu.make_async_copy(y_hbm.at[hgat[b]], yrows[b], sgy[b]).wait()
                pltpu.make_async_copy(t_hbm.at[tgat[b]], trows[b], sgt[b]).wait()

                def group(g, m):
                    vvec = jnp.zeros((L,), jnp.float32)
                    for e16 in range(L):
                        e = g * L + e16
                        acc = trows[b][e, pl.ds(0, L)] * yrows[b][e, pl.ds(0, L)]
                        for k in range(1, NKB):
                            acc = acc + trows[b][e, pl.ds(k * L, L)] * yrows[b][e, pl.ds(k * L, L)]
                        accs = _lane_rotsum(acc, lanes)
                        vvec = jnp.where(lanes == e16, accs, vvec)
                    r16 = ridx[b][pl.ds(g * L, L)]
                    svvec = svv[pl.ds(0, L)]
                    sv = jnp.zeros((L,), jnp.float32)
                    for r in range(N_REL):
                        sv = jnp.where(r16 == r, svvec[r], sv)
                    vvec = vvec + sv
                    vbuf[pl.ds(g * L, L)] = vvec
                    return jnp.maximum(m, vvec)
                m0 = mrow[0, pl.ds(0, L)]
                m = plsc.parallel_loop(0, CH // L, unroll=2, carry=m0)(group)
                mrow[0, pl.ds(0, L)] = m
                pltpu.sync_copy(vbuf, v_hbm.at[pl.ds(base, CH)])
        return 0
    lax.fori_loop(0, CPT // 2, pair, 0)
    m = mrow[0, pl.ds(0, L)]
    for i in range(8):
        mrow[i, pl.ds(0, L)] = m
    pltpu.sync_copy(mrow, mx_hbm.at[w])


def _expv_body(v_ref, mx_ref, ex_ref):
    m = jnp.max(mx_ref[...])
    ex_ref[...] = jnp.exp(v_ref[...] - m)


def _expv(v2d, mx):
    return pl.pallas_call(
        _expv_body,
        grid=(1,),
        in_specs=[
            pl.BlockSpec((E // DIM, DIM), lambda i: (0, 0)),
            pl.BlockSpec((NW * 8, L), lambda i: (0, 0)),
        ],
        out_specs=pl.BlockSpec((E // DIM, DIM), lambda i: (0, 0)),
        out_shape=jax.ShapeDtypeStruct((E // DIM, DIM), jnp.float32),
    )(v2d, mx)


@functools.partial(
    pl.kernel,
    out_type=[
        jax.ShapeDtypeStruct((NC, HALF, DIM), jnp.float32),
        jax.ShapeDtypeStruct((NC, HALF, DIM), jnp.float32),
    ],
    mesh=_mesh,
    scratch_types=[
        pltpu.VMEM((CH,), jnp.int32),
        pltpu.VMEM((CH,), jnp.int32),
        pltpu.VMEM((CH,), jnp.float32),
        pltpu.VMEM((CH, DIM), jnp.float32),
        pltpu.VMEM((CH, DIM), jnp.float32),
        pltpu.VMEM((RCH, DIM), jnp.float32),
        pltpu.VMEM((RCH, DIM), jnp.float32),
        pltpu.VMEM_SHARED((ACC_ROWS, DIM), jnp.float32),
        pltpu.VMEM_SHARED((ACC_ROWS, DIM), jnp.float32),
        pltpu.SemaphoreType.DMA,
    ],
)
def _hop2(ego1_hbm, hl_hbm, tl_hbm, ex_hbm,
          e2_hbm, den_hbm,
          hidx, tidx, vbuf, rows, db, zbuf, zbufd, acc, dacc, sem):
    c, s, w = _worker_ids()
    _zero_vmem(zbuf, RCH)
    _zero_vmem(zbufd, RCH)
    _zero_vmem(db, CH)

    def zero(k, _):
        rc = s + NS * k
        pltpu.sync_copy(zbuf, acc.at[pl.ds(rc * RCH, RCH)])
        pltpu.sync_copy(zbufd, dacc.at[pl.ds(rc * RCH, RCH)])
        return 0
    lax.fori_loop(0, RCPT, zero, 0)
    plsc.subcore_barrier()
    lo = c * HALF
    dump = HALF + s

    def chunk(j, _):
        cid = s + NS * j

        @pl.when(cid < NCHUNK)
        def _():
            base = cid * CH
            pltpu.sync_copy(ex_hbm.at[pl.ds(base, CH)], vbuf)
            pltpu.sync_copy(hl_hbm.at[pl.ds(base, CH)], hidx)
            pltpu.sync_copy(tl_hbm.at[pl.ds(base, CH)], tidx)
            for k in range(CH // L):
                hloc = hidx[pl.ds(k * L, L)] - lo
                ok = (hloc >= 0) & (hloc < HALF)
                hidx[pl.ds(k * L, L)] = jnp.where(ok, hloc, dump)
            pltpu.async_copy(ego1_hbm.at[tidx], rows, sem).wait()

            @plsc.parallel_loop(0, CH // L, unroll=2)
            def scale(g):
                x16 = vbuf[pl.ds(g * L, L)]
                for e16 in range(L):
                    x = x16[e16]
                    e = g * L + e16
                    xv = jnp.full((L,), x, jnp.float32)
                    db[e, pl.ds(0, L)] = xv
                    for k in range(NKB):
                        rows[e, pl.ds(k * L, L)] = rows[e, pl.ds(k * L, L)] * xv
            pltpu.sync_copy(rows, acc.at[hidx], add=True)
            pltpu.sync_copy(db, dacc.at[hidx], add=True)
        return 0
    lax.fori_loop(0, CPT1, chunk, 0)
    plsc.subcore_barrier()

    def wb(k, _):
        rc = s + NS * k

        @pl.when(rc < NRCH)
        def _():
            r0 = rc * RCH
            pltpu.sync_copy(acc.at[pl.ds(r0, RCH)], zbuf)
            pltpu.sync_copy(zbuf, e2_hbm.at[c, pl.ds(r0, RCH)])
            pltpu.sync_copy(dacc.at[pl.ds(r0, RCH)], zbufd)
            pltpu.sync_copy(zbufd, den_hbm.at[c, pl.ds(r0, RCH)])
        return 0
    lax.fori_loop(0, RCPT, wb, 0)


def _normkg_body(ego_ref, e2_ref, den_ref, kg_ref):
    d = den_ref[...][:, 0:1]
    d = jnp.where(d == 0.0, 1.0, d)
    kg_ref[...] = (ego_ref[...] + e2_ref[...] / d) * 0.5


def _normkg(ego1, e2, den):
    M = 1000
    return pl.pallas_call(
        _normkg_body,
        grid=(N_NODE // M,),
        in_specs=[
            pl.BlockSpec((M, DIM), lambda i: (i, 0)),
            pl.BlockSpec((M, DIM), lambda i: (i, 0)),
            pl.BlockSpec((M, DIM), lambda i: (i, 0)),
        ],
        out_specs=pl.BlockSpec((M, DIM), lambda i: (i, 0)),
        out_shape=jax.ShapeDtypeStruct((N_NODE, DIM), jnp.float32),
    )(ego1, e2, den)


NIDX = BS * SQ
IPT = NIDX // NW
ICH = 64


@functools.partial(
    pl.kernel,
    out_type=[
        jax.ShapeDtypeStruct((NIDX, DIM), jnp.float32),
        jax.ShapeDtypeStruct((NIDX, DIM), jnp.float32),
    ],
    mesh=_mesh,
    scratch_types=[
        pltpu.VMEM((ICH,), jnp.int32),
        pltpu.VMEM((ICH, DIM), jnp.float32),
        pltpu.VMEM((ICH, DIM), jnp.float32),
        pltpu.SemaphoreType.DMA,
        pltpu.SemaphoreType.DMA,
    ],
)
def _assemble(emb_hbm, kg_hbm,
              idx_hbm, hout_hbm, kgout_hbm,
              idxv, erows, grows, sem1, sem2):
    c, s, w = _worker_ids()

    def chunk(i, _):
        base = w * IPT + i * ICH
        pltpu.sync_copy(idx_hbm.at[pl.ds(base, ICH)], idxv)
        cp1 = pltpu.async_copy(emb_hbm.at[idxv], erows, sem1)
        cp2 = pltpu.async_copy(kg_hbm.at[idxv], grows, sem2)
        cp1.wait()
        cp2.wait()
        pltpu.sync_copy(erows, hout_hbm.at[pl.ds(base, ICH)])
        pltpu.sync_copy(grows, kgout_hbm.at[pl.ds(base, ICH)])
        return 0
    lax.fori_loop(0, IPT // ICH, chunk, 0)


BBLK = 8


def _dense_body(h_ref, kg_ref, a_ref, av_ref, w3_ref, out_ref):
    h = h_ref[...]
    kg = kg_ref[...]
    Ab = a_ref[...]
    av = av_ref[...]
    w3 = w3_ref[...]

    P = h[:, :, None, :] * h[:, None, :, :]
    P = jnp.where(P > 0, P, ALPHA * P)
    e4 = jnp.dot(P.reshape(BBLK * SQ * SQ, DIM), av,
                 preferred_element_type=jnp.float32)
    e4 = e4.reshape(BBLK, SQ, SQ, 4)

    big = jnp.float32(-9e15)
    al = jnp.full((BBLK, SQ, SQ), big, jnp.float32)
    for k in range(4):
        al = jnp.where(Ab == (k + 1), e4[..., k], al)
    al = al - jnp.max(al, axis=-1, keepdims=True)
    al = jnp.exp(al)
    al = al / jnp.sum(al, axis=-1, keepdims=True)

    hid = []
    for b in range(BBLK):
        hid.append(jnp.dot(al[b], h[b], preferred_element_type=jnp.float32))
    hidden = jnp.stack(hid)

    out = jnp.dot(hidden.reshape(BBLK * SQ, DIM), w3[:DIM],
                  preferred_element_type=jnp.float32)
    out = out + jnp.dot(kg.reshape(BBLK * SQ, DIM), w3[DIM:],
                        preferred_element_type=jnp.float32)
    out_ref[...] = jnp.tanh(out).reshape(BBLK, SQ, DIM)


def _dense(hrows, kgrows, A, av, w3):
    grid = (BS // BBLK,)
    return pl.pallas_call(
        _dense_body,
        grid=grid,
        in_specs=[
            pl.BlockSpec((BBLK, SQ, DIM), lambda i: (i, 0, 0)),
            pl.BlockSpec((BBLK, SQ, DIM), lambda i: (i, 0, 0)),
            pl.BlockSpec((BBLK, SQ, SQ), lambda i: (i, 0, 0)),
            pl.BlockSpec((DIM, 4), lambda i: (0, 0)),
            pl.BlockSpec((2 * DIM, DIM), lambda i: (0, 0)),
        ],
        out_specs=pl.BlockSpec((BBLK, SQ, DIM), lambda i: (i, 0, 0)),
        out_shape=jax.ShapeDtypeStruct((BS, SQ, DIM), jnp.float32),
    )(hrows, kgrows, A, av, w3)


def _xla_rest(inputs, A, emb, r_embed, trans_M, hl, tl, rl, A_vals,
              a_0, a_1, a_2, a_3, w_3, ego1, v=None, mx=None, kg=None):
    if v is None:
        Y = jnp.einsum('nd,rde->rne', ego1, trans_M)
        T = jnp.tanh(Y)
        sv = jnp.sum(r_embed, axis=1)
        yf = Y.reshape(N_REL * N_NODE, DIM)
        tf = T.reshape(N_REL * N_NODE, DIM)
        v = jnp.sum(tf[rl * N_NODE + tl] * yf[rl * N_NODE + hl], axis=1) + sv[rl]
    if kg is None:
        gm = jnp.max(v) if mx is None else jnp.max(mx)
        ex = jnp.exp(v - gm)
        den = jax.ops.segment_sum(ex, hl, num_segments=N_NODE)
        e2num = jax.ops.segment_sum(ex[:, None] * ego1[tl], hl, num_segments=N_NODE)
        dv = jnp.where(den == 0, 1., den)
        kg = (ego1 + e2num / dv[:, None]) * 0.5
    idx = inputs.astype(jnp.int32).reshape(-1)
    h = emb[idx].reshape(BS, SQ, DIM)
    kgr = kg[idx].reshape(BS, SQ, DIM)
    ai = h[:, :, None, :] * h[:, None, :, :]
    ai = jnp.where(ai > 0, ai, ALPHA * ai)
    av4 = jnp.concatenate([a_0, a_1, a_2, a_3], axis=1)
    e4 = (ai.reshape(-1, DIM) @ av4).reshape(BS, SQ, SQ, 4)
    al = jnp.full((BS, SQ, SQ), -9e15)
    for k in range(4):
        al = jnp.where(A == k + 1, e4[..., k], al)
    al = jax.nn.softmax(al, axis=-1)
    hid = jnp.einsum('bij,bjd->bid', al, h)
    return jnp.tanh(jnp.concatenate([hid, kgr], axis=-1) @ w_3)


def kernel(inputs, A, mask_item, embedding, r_embed, trans_M, h_list,
           t_list, r_list, A_vals, a_0, a_1, a_2, a_3, w_3):
    hl = h_list.astype(jnp.int32)
    tl = t_list.astype(jnp.int32)
    rl = r_list.astype(jnp.int32)
    emb = embedding.astype(jnp.float32)

    ego1 = _spmm1(emb, hl, tl, A_vals).reshape(N_NODE, DIM)
    Y, T = _relproj(ego1, trans_M)
    yflat = Y.reshape(N_REL * N_NODE, DIM)
    tflat = T.reshape(N_REL * N_NODE, DIM)
    svec = jnp.zeros((L,), jnp.float32).at[:N_REL].set(jnp.sum(r_embed, axis=1))
    v, mx = _edgescore(yflat, tflat, hl, tl, rl, svec)
    ex = _expv(v.reshape(E // DIM, DIM), mx.reshape(NW * 8, L)).reshape(E)
    e2, den = _hop2(ego1, hl, tl, ex)
    kg = _normkg(ego1, e2.reshape(N_NODE, DIM), den.reshape(N_NODE, DIM))
    idx = inputs.astype(jnp.int32).reshape(-1)
    hrows, kgrows = _assemble(emb, kg, idx)
    av = jnp.concatenate([a_0, a_1, a_2, a_3], axis=1)
    return _dense(hrows.reshape(BS, SQ, DIM), kgrows.reshape(BS, SQ, DIM),
                  A.astype(jnp.int32), av, w_3)
    Y, T = _relproj(ego1, trans_M)
    yflat = Y.reshape(N_REL * N_NODE, DIM)
    tflat = T.reshape(N_REL * N_NODE, DIM)

    svec = jnp.zeros((L,), jnp.float32).at[:N_REL].set(jnp.sum(r_embed, axis=1))
    v, mx = _edgescore(yflat, tflat, hl, tl, rl, svec)
    e2, den = _hop2(ego1, hl, tl, v, mx.reshape(NW * 8, L))

    kg = _normkg(ego1, e2.reshape(N_NODE, DIM), den.reshape(N_NODE, DIM))
    idx = inputs.astype(jnp.int32).reshape(-1)
    hrows, kgrows = _assemble(emb, kg, idx)

    av = jnp.concatenate([a_0, a_1, a_2, a_3], axis=1)
    out = _dense(hrows.reshape(BS, SQ, DIM), kgrows.reshape(BS, SQ, DIM),
                 A.astype(jnp.int32), av, w_3)
    return out


# speedup vs baseline: 4.6608x; 1.0006x over previous
import functools

import jax
import jax.numpy as jnp
from jax import lax
from jax.experimental import pallas as pl
from jax.experimental.pallas import tpu as pltpu
from jax.experimental.pallas import tpu_sc as plsc

N_NODE = 10000
N_REL = 8
E = 160000
DIM = 128
BS = 512
SQ = 20
ALPHA = 0.2

NC = 2
NS = 16
NW = NC * NS
L = 16

CH = 128
NCHUNK = E // CH
CPT = (NCHUNK + NW - 1) // NW
CPT1 = (NCHUNK + NS - 1) // NS
NKB = DIM // L

HALF = N_NODE // NC
ACC_ROWS = 5120
RCH = 40
NRCH = HALF // RCH
NZCH = ACC_ROWS // RCH
RCPT = NZCH // NS

_mesh = plsc.VectorSubcoreMesh(core_axis_name="c", subcore_axis_name="s")


def _worker_ids():
    c = lax.axis_index("c")
    s = lax.axis_index("s")
    return c, s, c * NS + s


def _lane_rotsum(x, lanes):
    for sh in (1, 2, 4, 8):
        idx = jnp.bitwise_and(lanes + sh, L - 1)
        x = x + x[idx]
    return x


def _zero_vmem(ref, nrows):
    def body(i, _):
        for k in range(ref.shape[1] // L):
            ref[i, pl.ds(k * L, L)] = jnp.zeros((L,), jnp.float32)
        return 0
    lax.fori_loop(0, nrows, body, 0)


NTRI = (CPT1 + 3) // 3


@functools.partial(
    pl.kernel,
    out_type=jax.ShapeDtypeStruct((NC, HALF, DIM), jnp.float32),
    mesh=_mesh,
    scratch_types=[
        [pltpu.VMEM((CH,), jnp.int32) for _ in range(3)],
        [pltpu.VMEM((CH,), jnp.int32) for _ in range(3)],
        pltpu.VMEM((CH,), jnp.float32),
        [pltpu.VMEM((CH, DIM), jnp.float32) for _ in range(3)],
        pltpu.VMEM((RCH, DIM), jnp.float32),
        pltpu.VMEM_SHARED((ACC_ROWS, DIM), jnp.float32),
        [pltpu.SemaphoreType.DMA for _ in range(3)],
        [pltpu.SemaphoreType.DMA for _ in range(3)],
    ],
)
def _spmm1(emb_hbm, hl_hbm, tl_hbm, av_hbm, out_hbm,
           tb, hb, vals, rb, zbuf, acc, sg, ss):
    c, s, w = _worker_ids()
    _zero_vmem(zbuf, RCH)

    def zero(k, _):
        rc = s + NS * k
        pltpu.sync_copy(zbuf, acc.at[pl.ds(rc * RCH, RCH)])
        return 0
    lax.fori_loop(0, RCPT, zero, 0)
    plsc.subcore_barrier()
    lo = c * HALF
    dump = HALF + s

    def load_idx(j, b):
        base = (s + NS * j) * CH
        pltpu.sync_copy(tl_hbm.at[pl.ds(base, CH)], tb[b])
        pltpu.sync_copy(hl_hbm.at[pl.ds(base, CH)], hb[b])
        for k in range(CH // L):
            hloc = hb[b][pl.ds(k * L, L)] - lo
            ok = (hloc >= 0) & (hloc < HALF)
            hb[b][pl.ds(k * L, L)] = jnp.where(ok, hloc, dump)

    load_idx(0, 0)
    pltpu.async_copy(emb_hbm.at[tb[0]], rb[0], sg[0])

    def triple(j3, _):
        for b3 in range(3):
            j = j3 * 3 + b3
            cid = s + NS * j
            nb = (b3 + 1) % 3

            @pl.when(s + NS * (j + 1) < NCHUNK)
            def _():
                def prefetch():
                    load_idx(j + 1, nb)
                    pltpu.async_copy(emb_hbm.at[tb[nb]], rb[nb], sg[nb])
                if b3 == 2:
                    pltpu.make_async_copy(rb[nb], acc.at[hb[nb]], ss[nb]).wait()
                    prefetch()
                else:
                    @pl.when(j3 >= 1)
                    def _():
                        pltpu.make_async_copy(rb[nb], acc.at[hb[nb]], ss[nb]).wait()
                        prefetch()

                    @pl.when(j3 < 1)
                    def _():
                        prefetch()

            @pl.when(cid < NCHUNK)
            def _():
                base = cid * CH
                pltpu.sync_copy(av_hbm.at[pl.ds(base, CH)], vals)
                pltpu.make_async_copy(emb_hbm.at[tb[b3]], rb[b3], sg[b3]).wait()

                @plsc.parallel_loop(0, CH // L, unroll=2)
                def scale(g):
                    v16 = vals[pl.ds(g * L, L)]
                    for e16 in range(L):
                        v = v16[e16]
                        e = g * L + e16
                        for k in range(NKB):
                            rb[b3][e, pl.ds(k * L, L)] = rb[b3][e, pl.ds(k * L, L)] * v
                pltpu.async_copy(rb[b3], acc.at[hb[b3]], ss[b3], add=True)
        return 0
    lax.fori_loop(0, NTRI, triple, 0)
    for b in range(3):
        pltpu.make_async_copy(rb[b], acc.at[hb[b]], ss[b]).wait()
    plsc.subcore_barrier()

    def wb(k, _):
        rc = s + NS * k

        @pl.when(rc < NRCH)
        def _():
            r0 = rc * RCH
            pltpu.sync_copy(acc.at[pl.ds(r0, RCH)], zbuf)
            pltpu.sync_copy(zbuf, out_hbm.at[c, pl.ds(r0, RCH)])
        return 0
    lax.fori_loop(0, RCPT, wb, 0)


def _relproj_body(ego_ref, w_ref, y_ref, t_ref):
    ego = ego_ref[...]
    for r in range(N_REL):
        yr = jnp.dot(ego, w_ref[r], preferred_element_type=jnp.float32)
        y_ref[r] = yr
        t_ref[r] = jnp.tanh(yr)


def _relproj(ego1, trans_M):
    M = 1000
    grid = (N_NODE // M,)
    return pl.pallas_call(
        _relproj_body,
        grid=grid,
        in_specs=[
            pl.BlockSpec((M, DIM), lambda i: (i, 0)),
            pl.BlockSpec((N_REL, DIM, DIM), lambda i: (0, 0, 0)),
        ],
        out_specs=[
            pl.BlockSpec((N_REL, M, DIM), lambda i: (0, i, 0)),
            pl.BlockSpec((N_REL, M, DIM), lambda i: (0, i, 0)),
        ],
        out_shape=[
            jax.ShapeDtypeStruct((N_REL, N_NODE, DIM), jnp.float32),
            jax.ShapeDtypeStruct((N_REL, N_NODE, DIM), jnp.float32),
        ],
    )(ego1, trans_M)


@functools.partial(
    pl.kernel,
    out_type=[
        jax.ShapeDtypeStruct((E,), jnp.float32),
        jax.ShapeDtypeStruct((NW, 8, L), jnp.float32),
    ],
    mesh=_mesh,
    scratch_types=[
        pltpu.VMEM((CH,), jnp.int32),
        pltpu.VMEM((CH,), jnp.int32),
        [pltpu.VMEM((CH,), jnp.int32) for _ in range(2)],
        [pltpu.VMEM((CH,), jnp.int32) for _ in range(2)],
        [pltpu.VMEM((CH,), jnp.int32) for _ in range(2)],
        [pltpu.VMEM((CH, DIM), jnp.float32) for _ in range(2)],
        [pltpu.VMEM((CH, DIM), jnp.float32) for _ in range(2)],
        pltpu.VMEM((L,), jnp.float32),
        pltpu.VMEM((CH,), jnp.float32),
        pltpu.VMEM((8, L), jnp.float32),
        [pltpu.SemaphoreType.DMA for _ in range(2)],
        [pltpu.SemaphoreType.DMA for _ in range(2)],
    ],
)
def _edgescore(y_hbm, t_hbm, hl_hbm, tl_hbm, rl_hbm, sv_hbm,
               v_hbm, mx_hbm,
               hidx, tidx, ridx, hgat, tgat, yrows, trows, svv, vbuf,
               mrow, sgy, sgt):
    c, s, w = _worker_ids()
    pltpu.sync_copy(sv_hbm, svv)
    lanes = jnp.arange(L, dtype=jnp.int32)
    mrow[0, pl.ds(0, L)] = jnp.full((L,), -3e38, jnp.float32)

    def prefetch(j, b):
        base = (w + NW * j) * CH
        pltpu.sync_copy(hl_hbm.at[pl.ds(base, CH)], hidx)
        pltpu.sync_copy(tl_hbm.at[pl.ds(base, CH)], tidx)
        pltpu.sync_copy(rl_hbm.at[pl.ds(base, CH)], ridx[b])
        for k in range(CH // L):
            rr = ridx[b][pl.ds(k * L, L)] * N_NODE
            hgat[b][pl.ds(k * L, L)] = rr + hidx[pl.ds(k * L, L)]
            tgat[b][pl.ds(k * L, L)] = rr + tidx[pl.ds(k * L, L)]
        pltpu.async_copy(y_hbm.at[hgat[b]], yrows[b], sgy[b])
        pltpu.async_copy(t_hbm.at[tgat[b]], trows[b], sgt[b])

    prefetch(0, 0)

    def pair(j2, _):
        for b in range(2):
            j = j2 * 2 + b
            cid = w + NW * j
            nb = 1 - b

            @pl.when(w + NW * (j + 1) < NCHUNK)
            def _():
                prefetch(j + 1, nb)

            @pl.when(cid < NCHUNK)
            def do():
                base = cid * CH
                pltpu.make_async_copy(y_hbm.at[hgat[b]], yrows[b], sgy[b]).wait()
                pltpu.make_async_copy(t_hbm.at[tgat[b]], trows[b], sgt[b]).wait()

                def group(g, m):
                    vvec = jnp.zeros((L,), jnp.float32)
                    for e16 in range(L):
                        e = g * L + e16
                        acc = trows[b][e, pl.ds(0, L)] * yrows[b][e, pl.ds(0, L)]
                        for k in range(1, NKB):
                            acc = acc + trows[b][e, pl.ds(k * L, L)] * yrows[b][e, pl.ds(k * L, L)]
                        accs = _lane_rotsum(acc, lanes)
                        vvec = jnp.where(lanes == e16, accs, vvec)
                    r16 = ridx[b][pl.ds(g * L, L)]
                    svvec = svv[pl.ds(0, L)]
                    sv = jnp.zeros((L,), jnp.float32)
                    for r in range(N_REL):
                        sv = jnp.where(r16 == r, svvec[r], sv)
                    vvec = vvec + sv
                    vbuf[pl.ds(g * L, L)] = vvec
                    return jnp.maximum(m, vvec)
                m0 = mrow[0, pl.ds(0, L)]
                m = plsc.parallel_loop(0, CH // L, unroll=2, carry=m0)(group)
                mrow[0, pl.ds(0, L)] = m
                pltpu.sync_copy(vbuf, v_hbm.at[pl.ds(base, CH)])
        return 0
    lax.fori_loop(0, CPT // 2, pair, 0)
    m = mrow[0, pl.ds(0, L)]
    for i in range(8):
        mrow[i, pl.ds(0, L)] = m
    pltpu.sync_copy(mrow, mx_hbm.at[w])


def _expv_body(v_ref, mx_ref, ex_ref):
    m = jnp.max(mx_ref[...])
    ex_ref[...] = jnp.exp(v_ref[...] - m)


def _expv(v2d, mx):
    return pl.pallas_call(
        _expv_body,
        grid=(1,),
        in_specs=[
            pl.BlockSpec((E // DIM, DIM), lambda i: (0, 0)),
            pl.BlockSpec((NW * 8, L), lambda i: (0, 0)),
        ],
        out_specs=pl.BlockSpec((E // DIM, DIM), lambda i: (0, 0)),
        out_shape=jax.ShapeDtypeStruct((E // DIM, DIM), jnp.float32),
    )(v2d, mx)


@functools.partial(
    pl.kernel,
    out_type=[
        jax.ShapeDtypeStruct((NC, HALF, DIM), jnp.float32),
        jax.ShapeDtypeStruct((NC, HALF, DIM), jnp.float32),
    ],
    mesh=_mesh,
    scratch_types=[
        pltpu.VMEM((CH,), jnp.int32),
        pltpu.VMEM((CH,), jnp.int32),
        pltpu.VMEM((CH,), jnp.float32),
        pltpu.VMEM((CH, DIM), jnp.float32),
        pltpu.VMEM((CH, DIM), jnp.float32),
        pltpu.VMEM((RCH, DIM), jnp.float32),
        pltpu.VMEM((RCH, DIM), jnp.float32),
        pltpu.VMEM_SHARED((ACC_ROWS, DIM), jnp.float32),
        pltpu.VMEM_SHARED((ACC_ROWS, DIM), jnp.float32),
        pltpu.SemaphoreType.DMA,
    ],
)
def _hop2(ego1_hbm, hl_hbm, tl_hbm, ex_hbm,
          e2_hbm, den_hbm,
          hidx, tidx, vbuf, rows, db, zbuf, zbufd, acc, dacc, sem):
    c, s, w = _worker_ids()
    _zero_vmem(zbuf, RCH)
    _zero_vmem(zbufd, RCH)
    _zero_vmem(db, CH)

    def zero(k, _):
        rc = s + NS * k
        pltpu.sync_copy(zbuf, acc.at[pl.ds(rc * RCH, RCH)])
        pltpu.sync_copy(zbufd, dacc.at[pl.ds(rc * RCH, RCH)])
        return 0
    lax.fori_loop(0, RCPT, zero, 0)
    plsc.subcore_barrier()
    lo = c * HALF
    dump = HALF + s

    def chunk(j, _):
        cid = s + NS * j

        @pl.when(cid < NCHUNK)
        def _():
            base = cid * CH
            pltpu.sync_copy(ex_hbm.at[pl.ds(base, CH)], vbuf)
            pltpu.sync_copy(hl_hbm.at[pl.ds(base, CH)], hidx)
            pltpu.sync_copy(tl_hbm.at[pl.ds(base, CH)], tidx)
            for k in range(CH // L):
                hloc = hidx[pl.ds(k * L, L)] - lo
                ok = (hloc >= 0) & (hloc < HALF)
                hidx[pl.ds(k * L, L)] = jnp.where(ok, hloc, dump)
            pltpu.async_copy(ego1_hbm.at[tidx], rows, sem).wait()

            @plsc.parallel_loop(0, CH // L, unroll=2)
            def scale(g):
                x16 = vbuf[pl.ds(g * L, L)]
                for e16 in range(L):
                    x = x16[e16]
                    e = g * L + e16
                    xv = jnp.full((L,), x, jnp.float32)
                    db[e, pl.ds(0, L)] = xv
                    for k in range(NKB):
                        rows[e, pl.ds(k * L, L)] = rows[e, pl.ds(k * L, L)] * xv
            pltpu.sync_copy(rows, acc.at[hidx], add=True)
            pltpu.sync_copy(db, dacc.at[hidx], add=True)
        return 0
    lax.fori_loop(0, CPT1, chunk, 0)
    plsc.subcore_barrier()

    def wb(k, _):
        rc = s + NS * k

        @pl.when(rc < NRCH)
        def _():
            r0 = rc * RCH
            pltpu.sync_copy(acc.at[pl.ds(r0, RCH)], zbuf)
            pltpu.sync_copy(zbuf, e2_hbm.at[c, pl.ds(r0, RCH)])
            pltpu.sync_copy(dacc.at[pl.ds(r0, RCH)], zbufd)
            pltpu.sync_copy(zbufd, den_hbm.at[c, pl.ds(r0, RCH)])
        return 0
    lax.fori_loop(0, RCPT, wb, 0)


def _normkg_body(ego_ref, e2_ref, den_ref, kg_ref):
    d = den_ref[...][:, 0:1]
    d = jnp.where(d == 0.0, 1.0, d)
    kg_ref[...] = (ego_ref[...] + e2_ref[...] / d) * 0.5


def _normkg(ego1, e2, den):
    M = 1000
    return pl.pallas_call(
        _normkg_body,
        grid=(N_NODE // M,),
        in_specs=[
            pl.BlockSpec((M, DIM), lambda i: (i, 0)),
            pl.BlockSpec((M, DIM), lambda i: (i, 0)),
            pl.BlockSpec((M, DIM), lambda i: (i, 0)),
        ],
        out_specs=pl.BlockSpec((M, DIM), lambda i: (i, 0)),
        out_shape=jax.ShapeDtypeStruct((N_NODE, DIM), jnp.float32),
    )(ego1, e2, den)


NIDX = BS * SQ
IPT = NIDX // NW
ICH = 64


@functools.partial(
    pl.kernel,
    out_type=[
        jax.ShapeDtypeStruct((NIDX, DIM), jnp.float32),
        jax.ShapeDtypeStruct((NIDX, DIM), jnp.float32),
    ],
    mesh=_mesh,
    scratch_types=[
        pltpu.VMEM((ICH,), jnp.int32),
        pltpu.VMEM((ICH, DIM), jnp.float32),
        pltpu.VMEM((ICH, DIM), jnp.float32),
        pltpu.SemaphoreType.DMA,
        pltpu.SemaphoreType.DMA,
    ],
)
def _assemble(emb_hbm, kg_hbm,
              idx_hbm, hout_hbm, kgout_hbm,
              idxv, erows, grows, sem1, sem2):
    c, s, w = _worker_ids()

    def chunk(i, _):
        base = w * IPT + i * ICH
        pltpu.sync_copy(idx_hbm.at[pl.ds(base, ICH)], idxv)
        cp1 = pltpu.async_copy(emb_hbm.at[idxv], erows, sem1)
        cp2 = pltpu.async_copy(kg_hbm.at[idxv], grows, sem2)
        cp1.wait()
        cp2.wait()
        pltpu.sync_copy(erows, hout_hbm.at[pl.ds(base, ICH)])
        pltpu.sync_copy(grows, kgout_hbm.at[pl.ds(base, ICH)])
        return 0
    lax.fori_loop(0, IPT // ICH, chunk, 0)


BBLK = 8


def _dense_body(h_ref, kg_ref, a_ref, av_ref, w3_ref, out_ref):
    h = h_ref[...]
    kg = kg_ref[...]
    Ab = a_ref[...]
    av = av_ref[...]
    w3 = w3_ref[...]

    P = h[:, :, None, :] * h[:, None, :, :]
    P = jnp.where(P > 0, P, ALPHA * P)
    e4 = jnp.dot(P.reshape(BBLK * SQ * SQ, DIM), av,
                 preferred_element_type=jnp.float32)
    e4 = e4.reshape(BBLK, SQ, SQ, 4)

    big = jnp.float32(-9e15)
    al = jnp.full((BBLK, SQ, SQ), big, jnp.float32)
    for k in range(4):
        al = jnp.where(Ab == (k + 1), e4[..., k], al)
    al = al - jnp.max(al, axis=-1, keepdims=True)
    al = jnp.exp(al)
    al = al / jnp.sum(al, axis=-1, keepdims=True)

    hid = []
    for b in range(BBLK):
        hid.append(jnp.dot(al[b], h[b], preferred_element_type=jnp.float32))
    hidden = jnp.stack(hid)

    out = jnp.dot(hidden.reshape(BBLK * SQ, DIM), w3[:DIM],
                  preferred_element_type=jnp.float32)
    out = out + jnp.dot(kg.reshape(BBLK * SQ, DIM), w3[DIM:],
                        preferred_element_type=jnp.float32)
    out_ref[...] = jnp.tanh(out).reshape(BBLK, SQ, DIM)


def _dense(hrows, kgrows, A, av, w3):
    grid = (BS // BBLK,)
    return pl.pallas_call(
        _dense_body,
        grid=grid,
        in_specs=[
            pl.BlockSpec((BBLK, SQ, DIM), lambda i: (i, 0, 0)),
            pl.BlockSpec((BBLK, SQ, DIM), lambda i: (i, 0, 0)),
            pl.BlockSpec((BBLK, SQ, SQ), lambda i: (i, 0, 0)),
            pl.BlockSpec((DIM, 4), lambda i: (0, 0)),
            pl.BlockSpec((2 * DIM, DIM), lambda i: (0, 0)),
        ],
        out_specs=pl.BlockSpec((BBLK, SQ, DIM), lambda i: (i, 0, 0)),
        out_shape=jax.ShapeDtypeStruct((BS, SQ, DIM), jnp.float32),
    )(hrows, kgrows, A, av, w3)


def kernel(inputs, A, mask_item, embedding, r_embed, trans_M, h_list,
           t_list, r_list, A_vals, a_0, a_1, a_2, a_3, w_3):
    hl = h_list.astype(jnp.int32)
    tl = t_list.astype(jnp.int32)
    rl = r_list.astype(jnp.int32)
    emb = embedding.astype(jnp.float32)

    ego1 = _spmm1(emb, hl, tl, A_vals).reshape(N_NODE, DIM)
    Y, T = _relproj(ego1, trans_M)
    yflat = Y.reshape(N_REL * N_NODE, DIM)
    tflat = T.reshape(N_REL * N_NODE, DIM)
    svec = jnp.zeros((L,), jnp.float32).at[:N_REL].set(jnp.sum(r_embed, axis=1))
    v, mx = _edgescore(yflat, tflat, hl, tl, rl, svec)
    ex = _expv(v.reshape(E // DIM, DIM), mx.reshape(NW * 8, L)).reshape(E)
    e2, den = _hop2(ego1, hl, tl, ex)
    kg = _normkg(ego1, e2.reshape(N_NODE, DIM), den.reshape(N_NODE, DIM))
    idx = inputs.astype(jnp.int32).reshape(-1)
    hrows, kgrows = _assemble(emb, kg, idx)
    av = jnp.concatenate([a_0, a_1, a_2, a_3], axis=1)
    return _dense(hrows.reshape(BS, SQ, DIM), kgrows.reshape(BS, SQ, DIM),
                  A.astype(jnp.int32), av, w_3)
    Y, T = _relproj(ego1, trans_M)
    yflat = Y.reshape(N_REL * N_NODE, DIM)
    tflat = T.reshape(N_REL * N_NODE, DIM)

    svec = jnp.zeros((L,), jnp.float32).at[:N_REL].set(jnp.sum(r_embed, axis=1))
    v, mx = _edgescore(yflat, tflat, hl, tl, rl, svec)
    e2, den = _hop2(ego1, hl, tl, v, mx.reshape(NW * 8, L))

    kg = _normkg(ego1, e2.reshape(N_NODE, DIM), den.reshape(N_NODE, DIM))
    idx = inputs.astype(jnp.int32).reshape(-1)
    hrows, kgrows = _assemble(emb, kg, idx)

    av = jnp.concatenate([a_0, a_1, a_2, a_3], axis=1)
    out = _dense(hrows.reshape(BS, SQ, DIM), kgrows.reshape(BS, SQ, DIM),
                 A.astype(jnp.int32), av, w_3)
    return out
